# Initial kernel scaffold; baseline (speedup 1.0000x reference)
import jax
import jax.numpy as jnp
from jax.experimental import pallas as pl


def kernel(x, batch, G_sim, G_sim_weight, gene_emb, pert_emb, sg_W0, sg_b0, sg_W1, sg_b1, rec_W1, rec_b1, rec_W2, rec_b2, rec_W3, rec_b3):
    raise NotImplementedError("write your pallas kernel here")


# speedup vs baseline: 7.6114x; 7.6114x over previous
import functools

import jax
import jax.numpy as jnp
from jax import lax
from jax.experimental import pallas as pl
from jax.experimental.pallas import tpu as pltpu
from jax.experimental.pallas import tpu_sc as plsc

NG = 10000
NGP = 10240
HID = 128
G = 8
E = 320000
CHUNK = 80
EPS = 1e-5
NORM_EPS = 1e-7
LAM = 0.2

_MESH = dict(core_axis_name="c", subcore_axis_name="s")


def _rsqrt_nr(x):
    i = lax.bitcast_convert_type(x, jnp.int32)
    i = jnp.int32(0x5F3759DF) - lax.shift_right_arithmetic(i, 1)
    y = lax.bitcast_convert_type(i, jnp.float32)
    for _ in range(3):
        y = y * (1.5 - 0.5 * x * y * y)
    return y


def _sc_deg_dinv(dst, ew):
    ept = E // 16
    nchunks = ept // CHUNK

    @functools.partial(
        pl.kernel,
        out_type=jax.ShapeDtypeStruct((NGP,), jnp.float32),
        mesh=plsc.VectorSubcoreMesh(**_MESH),
        scratch_types=[
            pltpu.VMEM_SHARED((NGP,), jnp.float32),
            pltpu.VMEM((CHUNK,), jnp.int32),
            pltpu.VMEM((CHUNK,), jnp.float32),
            pltpu.VMEM((640,), jnp.float32),
            pltpu.VMEM((320,), jnp.float32),
        ],
    )
    def deg_kernel(dst_hbm, ew_hbm, dinv_hbm, acc_sh, dst_v, ew_v, one_v, dbuf_v):
        c = lax.axis_index("c")
        s = lax.axis_index("s")

        @pl.loop(0, 40)
        def _fill(i):
            one_v[pl.ds(i * 16, 16)] = jnp.ones((16,), jnp.float32)

        pltpu.sync_copy(one_v, acc_sh.at[pl.ds(s * 640, 640)])
        plsc.subcore_barrier()

        @pl.loop(0, nchunks)
        def _edges(j):
            base = s * ept + j * CHUNK
            pltpu.sync_copy(dst_hbm.at[pl.ds(base, CHUNK)], dst_v)
            pltpu.sync_copy(ew_hbm.at[pl.ds(base, CHUNK)], ew_v)
            pltpu.sync_copy(ew_v, acc_sh.at[dst_v], add=True)

        plsc.subcore_barrier()
        row0 = c * 5120 + s * 320
        pltpu.sync_copy(acc_sh.at[pl.ds(row0, 320)], dbuf_v)

        @pl.loop(0, 20)
        def _rsq(i):
            sl = pl.ds(i * 16, 16)
            dbuf_v[sl] = _rsqrt_nr(dbuf_v[sl])

        pltpu.sync_copy(dbuf_v, dinv_hbm.at[pl.ds(row0, 320)])

    return deg_kernel(dst, ew)


def _sc_edge_scatter(src, dst, ew, table):
    ept = E // 32
    nchunks = ept // CHUNK
    rows_per_tile = NGP // 16

    @functools.partial(
        pl.kernel,
        out_type=jax.ShapeDtypeStruct((2, NGP, HID), jnp.float32),
        mesh=plsc.VectorSubcoreMesh(**_MESH),
        scratch_types=[
            pltpu.VMEM_SHARED((NGP, HID), jnp.float32),
            pltpu.VMEM((CHUNK,), jnp.int32),
            pltpu.VMEM((CHUNK,), jnp.int32),
            pltpu.VMEM((CHUNK,), jnp.float32),
            pltpu.VMEM((CHUNK, HID), jnp.float32),
            pltpu.VMEM((128, HID), jnp.float32),
            pltpu.SemaphoreType.DMA,
        ],
    )
    def scat_kernel(src_hbm, dst_hbm, ew_hbm, tab_hbm, out_hbm,
                    acc_sh, src_v, dst_v, ew_v, rows_v, zero_v, sem):
        c = lax.axis_index("c")
        s = lax.axis_index("s")
        wid = s * 2 + c

        @pl.loop(0, 128)
        def _z(r):
            for k in range(HID // 16):
                zero_v[r, pl.ds(k * 16, 16)] = jnp.zeros((16,), jnp.float32)

        for t in range(5):
            pltpu.sync_copy(zero_v, acc_sh.at[pl.ds(s * rows_per_tile + t * 128, 128)])
        plsc.subcore_barrier()

        @pl.loop(0, nchunks)
        def _edges(j):
            base = wid * ept + j * CHUNK
            pltpu.sync_copy(src_hbm.at[pl.ds(base, CHUNK)], src_v)
            pltpu.sync_copy(dst_hbm.at[pl.ds(base, CHUNK)], dst_v)
            pltpu.sync_copy(ew_hbm.at[pl.ds(base, CHUNK)], ew_v)
            pltpu.async_copy(tab_hbm.at[src_v], rows_v, sem).wait()

            @pl.loop(0, CHUNK // 16)
            def _scale(b):
                wv = ew_v[pl.ds(b * 16, 16)]
                for e in range(16):
                    w = wv[e]
                    for k in range(HID // 16):
                        sl = pl.ds(k * 16, 16)
                        rows_v[b * 16 + e, sl] = rows_v[b * 16 + e, sl] * w

            pltpu.sync_copy(rows_v, acc_sh.at[dst_v], add=True)

        plsc.subcore_barrier()
        r0 = s * rows_per_tile
        pltpu.sync_copy(acc_sh.at[pl.ds(r0, rows_per_tile)],
                        out_hbm.at[c, pl.ds(r0, rows_per_tile)])

    return scat_kernel(src, dst, ew, table)


def _k1_body(ge_ref, pe_ref, base_ref, p0_ref, bm_ref, bsq_ref):
    ge = ge_ref[...]
    n = jnp.sqrt(jnp.sum(ge * ge, axis=1, keepdims=True))
    r = ge * jnp.minimum(1.0, 1.0 / (n + NORM_EPS))
    m = jnp.mean(r, axis=0, keepdims=True)
    v = jnp.mean(r * r, axis=0, keepdims=True) - m * m
    base = jnp.maximum((r - m) / jnp.sqrt(v + EPS), 0.0)
    base_ref[...] = base
    bm_ref[...] = jnp.mean(base, axis=0, keepdims=True)
    bsq_ref[...] = jnp.mean(base * base, axis=0, keepdims=True)
    pe = pe_ref[...]
    n2 = jnp.sqrt(jnp.sum(pe * pe, axis=1, keepdims=True))
    p0_ref[...] = pe * jnp.minimum(1.0, 1.0 / (n2 + NORM_EPS))


def _tc_prep(gene_emb, pert_emb):
    return pl.pallas_call(
        _k1_body,
        out_shape=(
            jax.ShapeDtypeStruct((NG, HID), jnp.float32),
            jax.ShapeDtypeStruct((NG, HID), jnp.float32),
            jax.ShapeDtypeStruct((1, HID), jnp.float32),
            jax.ShapeDtypeStruct((1, HID), jnp.float32),
        ),
    )(gene_emb, pert_emb)


def _k5_body(sp_ref, xf_ref, dc_ref, w_ref, b_ref, out_ref, *, do_relu):
    dc = dc_ref[...]
    z = (sp_ref[0, :NG] + sp_ref[1, :NG]) * dc + xf_ref[...] * (dc * dc)
    h = jnp.dot(z, w_ref[...], preferred_element_type=jnp.float32) + b_ref[...]
    out_ref[...] = jnp.maximum(h, 0.0) if do_relu else h


def _tc_sg_linear(sp, xf, dinv_col, Wt, b, do_relu):
    return pl.pallas_call(
        functools.partial(_k5_body, do_relu=do_relu),
        out_shape=jax.ShapeDtypeStruct((NG, HID), jnp.float32),
    )(sp, xf, dinv_col, Wt, b.reshape(1, HID))


def _krs_body(t_ref, d_ref, out_ref):
    out_ref[...] = t_ref[...] * d_ref[...]


def _tc_rowscale(tab, dinv_col):
    return pl.pallas_call(
        _krs_body,
        out_shape=jax.ShapeDtypeStruct((NG, HID), jnp.float32),
    )(tab, dinv_col)


def _k7_body(pm_ref, pge_ref, add_ref):
    add_ref[...] = LAM * jnp.dot(pm_ref[...], pge_ref[...],
                                 preferred_element_type=jnp.float32)


def _tc_pert_mix(pertmat, pge2):
    return pl.pallas_call(
        _k7_body,
        out_shape=jax.ShapeDtypeStruct((G, HID), jnp.float32),
    )(pertmat, pge2)


def _k8_body(base_ref, w_ref, c_ref, p_ref, pm_ref, psq_ref):
    p = jnp.dot(base_ref[...], w_ref[...],
                preferred_element_type=jnp.float32) - c_ref[...]
    p_ref[...] = p
    pm_ref[...] = jnp.mean(p, axis=0, keepdims=True)
    psq_ref[...] = jnp.mean(p * p, axis=0, keepdims=True)


def _tc_layer1(base, W1s, c1):
    return pl.pallas_call(
        _k8_body,
        out_shape=(
            jax.ShapeDtypeStruct((NG, 2 * HID), jnp.float32),
            jax.ShapeDtypeStruct((1, 2 * HID), jnp.float32),
            jax.ShapeDtypeStruct((1, 2 * HID), jnp.float32),
        ),
    )(base, W1s, c1)


TR = 1000


def _k9_body(p_ref, al_ref, be_ref, w_ref, b_ref, h2_ref, hs_ref, hq_ref):
    first = jnp.logical_and(pl.program_id(0) == 0, pl.program_id(1) == 0)
    r = jnp.maximum(p_ref[...] * al_ref[...] + be_ref[0], 0.0)
    h2 = jnp.dot(r, w_ref[...], preferred_element_type=jnp.float32) + b_ref[...]
    h2_ref[...] = h2
    srow = jnp.sum(h2, axis=0, keepdims=True)
    qrow = jnp.sum(h2 * h2, axis=0, keepdims=True)

    @pl.when(first)
    def _():
        hs_ref[...] = srow
        hq_ref[...] = qrow

    @pl.when(jnp.logical_not(first))
    def _():
        hs_ref[...] = hs_ref[...] + srow
        hq_ref[...] = hq_ref[...] + qrow


def _tc_layer2(P, alpha, beta, W2t, b2):
    nt = NG // TR
    return pl.pallas_call(
        _k9_body,
        grid=(G, nt),
        in_specs=[
            pl.BlockSpec((TR, 2 * HID), lambda g, i: (i, 0)),
            pl.BlockSpec((1, 2 * HID), lambda g, i: (0, 0)),
            pl.BlockSpec((1, 1, 2 * HID), lambda g, i: (g, 0, 0)),
            pl.BlockSpec((2 * HID, HID), lambda g, i: (0, 0)),
            pl.BlockSpec((1, HID), lambda g, i: (0, 0)),
        ],
        out_specs=(
            pl.BlockSpec((TR, HID), lambda g, i: (g * nt + i, 0)),
            pl.BlockSpec((1, HID), lambda g, i: (0, 0)),
            pl.BlockSpec((1, HID), lambda g, i: (0, 0)),
        ),
        out_shape=(
            jax.ShapeDtypeStruct((G * NG, HID), jnp.float32),
            jax.ShapeDtypeStruct((1, HID), jnp.float32),
            jax.ShapeDtypeStruct((1, HID), jnp.float32),
        ),
    )(P, alpha, beta.reshape(G, 1, 2 * HID), W2t, b2)


def _k10_body(h2_ref, a_ref, c_ref, w_ref, b_ref, h3_ref, hs_ref, hq_ref):
    first = pl.program_id(0) == 0
    t = jnp.maximum(h2_ref[...] * a_ref[...] + c_ref[...], 0.0) * w_ref[...]
    row = jnp.sum(t, axis=1, keepdims=True) + b_ref[...]
    h3_ref[...] = row
    srow = jnp.sum(row, axis=0, keepdims=True)
    qrow = jnp.sum(row * row, axis=0, keepdims=True)

    @pl.when(first)
    def _():
        hs_ref[...] = srow
        hq_ref[...] = qrow

    @pl.when(jnp.logical_not(first))
    def _():
        hs_ref[...] = hs_ref[...] + srow
        hq_ref[...] = hq_ref[...] + qrow


def _tc_layer3(H2, a3, c3, w3, b3):
    nt = (G * NG) // TR
    return pl.pallas_call(
        _k10_body,
        grid=(nt,),
        in_specs=[
            pl.BlockSpec((TR, HID), lambda i: (i, 0)),
            pl.BlockSpec((1, HID), lambda i: (0, 0)),
            pl.BlockSpec((1, HID), lambda i: (0, 0)),
            pl.BlockSpec((1, HID), lambda i: (0, 0)),
            pl.BlockSpec((1, 1), lambda i: (0, 0)),
        ],
        out_specs=(
            pl.BlockSpec((TR, 1), lambda i: (i, 0)),
            pl.BlockSpec((1, 1), lambda i: (0, 0)),
            pl.BlockSpec((1, 1), lambda i: (0, 0)),
        ),
        out_shape=(
            jax.ShapeDtypeStruct((G * NG, 1), jnp.float32),
            jax.ShapeDtypeStruct((1, 1), jnp.float32),
            jax.ShapeDtypeStruct((1, 1), jnp.float32),
        ),
    )(H2, a3, c3, w3, b3)


def _k11_body(h3_ref, e_ref, a_ref, c_ref, out_ref):
    out_ref[...] = h3_ref[...] * a_ref[...] + c_ref[...] + e_ref[...]


def _tc_final(h3, expr, a4, c4):
    nt = (G * NG) // TR
    return pl.pallas_call(
        _k11_body,
        grid=(nt,),
        in_specs=[
            pl.BlockSpec((TR, 1), lambda i: (i, 0)),
            pl.BlockSpec((TR, 1), lambda i: (i, 0)),
            pl.BlockSpec((1, 1), lambda i: (0, 0)),
            pl.BlockSpec((1, 1), lambda i: (0, 0)),
        ],
        out_specs=pl.BlockSpec((TR, 1), lambda i: (i, 0)),
        out_shape=jax.ShapeDtypeStruct((G * NG, 1), jnp.float32),
    )(h3, expr, a4, c4)


def kernel(x, batch, G_sim, G_sim_weight, gene_emb, pert_emb,
           sg_W0, sg_b0, sg_W1, sg_b1,
           rec_W1, rec_b1, rec_W2, rec_b2, rec_W3, rec_b3):
    src = G_sim[0]
    dst = G_sim[1]

    base, p0, bm, bsq = _tc_prep(gene_emb, pert_emb)

    dinv = _sc_deg_dinv(dst, G_sim_weight)
    dinv_col = dinv[:NG].reshape(NG, 1)

    y0 = _tc_rowscale(p0, dinv_col)
    s0 = _sc_edge_scatter(src, dst, G_sim_weight, y0)
    pge1 = _tc_sg_linear(s0, p0, dinv_col, sg_W0.T, sg_b0, do_relu=True)
    y1 = _tc_rowscale(pge1, dinv_col)
    s1 = _sc_edge_scatter(src, dst, G_sim_weight, y1)
    pge2 = _tc_sg_linear(s1, pge1, dinv_col, sg_W1.T, sg_b1, do_relu=False)

    pertmat = x[:, 1].reshape(G, NG)
    add = _tc_pert_mix(pertmat, pge2)

    am = jnp.mean(add, axis=0, keepdims=True)
    av = jnp.mean(add * add, axis=0, keepdims=True) - am * am
    m1 = bm + am
    v1 = (bsq - bm * bm) + av
    sd1 = jnp.sqrt(v1 + EPS)

    W1s = rec_W1.T / sd1.reshape(HID, 1)
    c1 = (m1 / sd1) @ rec_W1.T
    Q = (add / sd1) @ rec_W1.T + rec_b1
    P, Pm, Psq = _tc_layer1(base, W1s, c1)
    qm = jnp.mean(Q, axis=0, keepdims=True)
    qv = jnp.mean(Q * Q, axis=0, keepdims=True) - qm * qm
    m2 = Pm + qm
    sd2 = jnp.sqrt((Psq - Pm * Pm) + qv + EPS)
    alpha = 1.0 / sd2
    beta = (Q - m2) / sd2

    H2, Hs, Hq = _tc_layer2(P, alpha, beta, rec_W2.T, rec_b2.reshape(1, HID))
    m3 = Hs / (G * NG)
    v3 = Hq / (G * NG) - m3 * m3
    sd3 = jnp.sqrt(v3 + EPS)
    a3 = 1.0 / sd3
    c3 = -m3 / sd3

    h3, hs, hq = _tc_layer3(H2, a3, c3, rec_W3, rec_b3.reshape(1, 1))
    m4 = hs / (G * NG)
    v4 = hq / (G * NG) - m4 * m4
    sd4 = jnp.sqrt(v4 + EPS)

    return _tc_final(h3, x[:, 0:1], 1.0 / sd4, -m4 / sd4)


# speedup vs baseline: 7.8501x; 1.0314x over previous
import functools

import jax
import jax.numpy as jnp
from jax import lax
from jax.experimental import pallas as pl
from jax.experimental.pallas import tpu as pltpu
from jax.experimental.pallas import tpu_sc as plsc

NG = 10000
NGP = 10240
HID = 128
G = 8
E = 320000
CHUNK = 80
EPS = 1e-5
NORM_EPS = 1e-7
LAM = 0.2

_MESH = dict(core_axis_name="c", subcore_axis_name="s")


EPAD = 2560 * 128
ECH = 128
CPTD = 80
CPT = 160
CW = HID // 2
RPT = NGP // 16


def _sc_deg(dst2, ew2):
    @functools.partial(
        pl.kernel,
        out_type=jax.ShapeDtypeStruct((2, NGP), jnp.float32),
        mesh=plsc.VectorSubcoreMesh(**_MESH),
        scratch_types=[
            pltpu.VMEM_SHARED((NGP,), jnp.float32),
            pltpu.VMEM((CPTD, ECH), jnp.int32),
            pltpu.VMEM((CPTD, ECH), jnp.float32),
            pltpu.VMEM((640,), jnp.float32),
            pltpu.SemaphoreType.DMA,
        ],
    )
    def deg_kernel(dst_hbm, ew_hbm, out_hbm, acc_sh, dst_v, ew_v, zero_v, sem):
        c = lax.axis_index("c")
        s = lax.axis_index("s")
        wid = s * 2 + c

        @pl.loop(0, 40)
        def _fill(i):
            zero_v[pl.ds(i * 16, 16)] = jnp.zeros((16,), jnp.float32)

        pltpu.sync_copy(zero_v, acc_sh.at[pl.ds(s * 640, 640)])
        pltpu.sync_copy(dst_hbm.at[pl.ds(wid * CPTD, CPTD)], dst_v)
        pltpu.sync_copy(ew_hbm.at[pl.ds(wid * CPTD, CPTD)], ew_v)
        plsc.subcore_barrier()

        @pl.loop(0, CPTD // 8)
        def _groups(t):
            ds_ = [
                pltpu.async_copy(ew_v.at[t * 8 + u], acc_sh.at[dst_v.at[t * 8 + u]],
                                 sem, add=True)
                for u in range(8)
            ]
            for d in ds_:
                d.wait()

        plsc.subcore_barrier()
        pltpu.sync_copy(acc_sh.at[pl.ds(s * 640, 640)],
                        out_hbm.at[c, pl.ds(s * 640, 640)])

    return deg_kernel(dst2, ew2)


def _sc_edge_scatter(src2, dst2, ew2, table):
    @functools.partial(
        pl.kernel,
        out_type=jax.ShapeDtypeStruct((2, NGP, HID), jnp.float32),
        mesh=plsc.VectorSubcoreMesh(**_MESH),
        scratch_types=[
            pltpu.VMEM_SHARED((NGP, HID), jnp.float32),
            pltpu.VMEM((CPTD, ECH), jnp.int32),
            pltpu.VMEM((2, ECH), jnp.int32),
            pltpu.VMEM((2, ECH), jnp.float32),
            pltpu.VMEM((2, ECH, HID), jnp.float32),
            [pltpu.SemaphoreType.DMA] * 2,
            [pltpu.SemaphoreType.DMA] * 2,
            [pltpu.SemaphoreType.DMA] * 2,
        ],
    )
    def scat_kernel(src_hbm, dst_hbm, ew_hbm, tab_hbm, out_hbm,
                    acc_sh, src_v, dst_v, ew_v, rows_v, isems, gsems, ssems):
        c = lax.axis_index("c")
        s = lax.axis_index("s")
        wid = s * 2 + c
        base = wid * CPTD

        @pl.loop(0, ECH)
        def _z(r):
            for k in range(HID // 16):
                rows_v[0, r, pl.ds(k * 16, 16)] = jnp.zeros((16,), jnp.float32)

        for t in range(RPT // ECH):
            pltpu.sync_copy(rows_v.at[0], acc_sh.at[pl.ds(s * RPT + t * ECH, ECH)])
        pltpu.sync_copy(src_hbm.at[pl.ds(base, CPTD)], src_v)
        plsc.subcore_barrier()

        def fire(j, u):
            pltpu.async_copy(dst_hbm.at[base + j], dst_v.at[u], isems[u])
            pltpu.async_copy(ew_hbm.at[base + j], ew_v.at[u], isems[u])
            pltpu.async_copy(tab_hbm.at[src_v.at[j]], rows_v.at[u], gsems[u])

        def wait_in(u):
            pltpu.make_async_copy(dst_hbm.at[0], dst_v.at[u], isems[u]).wait()
            pltpu.make_async_copy(ew_hbm.at[0], ew_v.at[u], isems[u]).wait()
            pltpu.make_async_copy(tab_hbm.at[src_v.at[0]], rows_v.at[u],
                                  gsems[u]).wait()

        def scale(u):
            @pl.loop(0, ECH // 16)
            def _scale(b):
                wv = ew_v[u, pl.ds(b * 16, 16)]
                for e in range(16):
                    w = wv[e]
                    for k in range(HID // 16):
                        sl = pl.ds(k * 16, 16)
                        rows_v[u, b * 16 + e, sl] = rows_v[u, b * 16 + e, sl] * w

        def scat(u):
            pltpu.async_copy(rows_v.at[u], acc_sh.at[dst_v.at[u]], ssems[u],
                             add=True)

        def wait_scat(u):
            pltpu.make_async_copy(rows_v.at[u], acc_sh.at[dst_v.at[u]],
                                  ssems[u]).wait()

        fire(0, 0)
        fire(1, 1)

        @pl.loop(0, CPTD // 2)
        def _pairs(t):
            j0 = t * 2
            wait_in(0)
            scale(0)
            scat(0)
            wait_in(1)
            scale(1)
            scat(1)
            wait_scat(0)

            @pl.when(j0 + 2 < CPTD)
            def _():
                fire(j0 + 2, 0)

            wait_scat(1)

            @pl.when(j0 + 3 < CPTD)
            def _():
                fire(j0 + 3, 1)

        plsc.subcore_barrier()
        pltpu.sync_copy(acc_sh.at[pl.ds(s * RPT, RPT)],
                        out_hbm.at[c, pl.ds(s * RPT, RPT)])

    return scat_kernel(src2, dst2, ew2, table)


def _k1_body(ge_ref, pe_ref, base_ref, p0_ref, bm_ref, bsq_ref):
    ge = ge_ref[...]
    n = jnp.sqrt(jnp.sum(ge * ge, axis=1, keepdims=True))
    r = ge * jnp.minimum(1.0, 1.0 / (n + NORM_EPS))
    m = jnp.mean(r, axis=0, keepdims=True)
    v = jnp.mean(r * r, axis=0, keepdims=True) - m * m
    base = jnp.maximum((r - m) / jnp.sqrt(v + EPS), 0.0)
    base_ref[...] = base
    bm_ref[...] = jnp.mean(base, axis=0, keepdims=True)
    bsq_ref[...] = jnp.mean(base * base, axis=0, keepdims=True)
    pe = pe_ref[...]
    n2 = jnp.sqrt(jnp.sum(pe * pe, axis=1, keepdims=True))
    p0_ref[...] = pe * jnp.minimum(1.0, 1.0 / (n2 + NORM_EPS))


def _tc_prep(gene_emb, pert_emb):
    return pl.pallas_call(
        _k1_body,
        out_shape=(
            jax.ShapeDtypeStruct((NG, HID), jnp.float32),
            jax.ShapeDtypeStruct((NG, HID), jnp.float32),
            jax.ShapeDtypeStruct((1, HID), jnp.float32),
            jax.ShapeDtypeStruct((1, HID), jnp.float32),
        ),
    )(gene_emb, pert_emb)


def _k5_body(sp_ref, xf_ref, dc_ref, w_ref, b_ref, out_ref, *, do_relu):
    dc = dc_ref[...]
    z = (sp_ref[0, :NG] + sp_ref[1, :NG]) * dc + xf_ref[...] * (dc * dc)
    h = jnp.dot(z, w_ref[...], preferred_element_type=jnp.float32) + b_ref[...]
    out_ref[...] = jnp.maximum(h, 0.0) if do_relu else h


def _tc_sg_linear(sp, xf, dinv_col, Wt, b, do_relu):
    return pl.pallas_call(
        functools.partial(_k5_body, do_relu=do_relu),
        out_shape=jax.ShapeDtypeStruct((NG, HID), jnp.float32),
    )(sp, xf, dinv_col, Wt, b.reshape(1, HID))


def _kdinv_body(dp_ref, out_ref):
    out_ref[...] = lax.rsqrt(dp_ref[0:1] + dp_ref[1:2] + 1.0)


def _tc_dinv(degp):
    return pl.pallas_call(
        _kdinv_body,
        out_shape=jax.ShapeDtypeStruct((1, NGP), jnp.float32),
    )(degp)


def _krs_body(t_ref, d_ref, out_ref):
    out_ref[...] = t_ref[...] * d_ref[...]


def _tc_rowscale(tab, dinv_col):
    return pl.pallas_call(
        _krs_body,
        out_shape=jax.ShapeDtypeStruct((NG, HID), jnp.float32),
    )(tab, dinv_col)


def _k7_body(pm_ref, pge_ref, add_ref):
    add_ref[...] = LAM * jnp.dot(pm_ref[...], pge_ref[...],
                                 preferred_element_type=jnp.float32)


def _tc_pert_mix(pertmat, pge2):
    return pl.pallas_call(
        _k7_body,
        out_shape=jax.ShapeDtypeStruct((G, HID), jnp.float32),
    )(pertmat, pge2)


def _k8_body(base_ref, w_ref, c_ref, p_ref, pm_ref, psq_ref):
    p = jnp.dot(base_ref[...], w_ref[...],
                preferred_element_type=jnp.float32) - c_ref[...]
    p_ref[...] = p
    pm_ref[...] = jnp.mean(p, axis=0, keepdims=True)
    psq_ref[...] = jnp.mean(p * p, axis=0, keepdims=True)


def _tc_layer1(base, W1s, c1):
    return pl.pallas_call(
        _k8_body,
        out_shape=(
            jax.ShapeDtypeStruct((NG, 2 * HID), jnp.float32),
            jax.ShapeDtypeStruct((1, 2 * HID), jnp.float32),
            jax.ShapeDtypeStruct((1, 2 * HID), jnp.float32),
        ),
    )(base, W1s, c1)


TR = 1000


def _k9_body(p_ref, al_ref, be_ref, w_ref, b_ref, h2_ref, hs_ref, hq_ref):
    first = jnp.logical_and(pl.program_id(0) == 0, pl.program_id(1) == 0)
    r = jnp.maximum(p_ref[...] * al_ref[...] + be_ref[0], 0.0)
    h2 = jnp.dot(r, w_ref[...], preferred_element_type=jnp.float32) + b_ref[...]
    h2_ref[...] = h2
    srow = jnp.sum(h2, axis=0, keepdims=True)
    qrow = jnp.sum(h2 * h2, axis=0, keepdims=True)

    @pl.when(first)
    def _():
        hs_ref[...] = srow
        hq_ref[...] = qrow

    @pl.when(jnp.logical_not(first))
    def _():
        hs_ref[...] = hs_ref[...] + srow
        hq_ref[...] = hq_ref[...] + qrow


def _tc_layer2(P, alpha, beta, W2t, b2):
    nt = NG // TR
    return pl.pallas_call(
        _k9_body,
        grid=(G, nt),
        in_specs=[
            pl.BlockSpec((TR, 2 * HID), lambda g, i: (i, 0)),
            pl.BlockSpec((1, 2 * HID), lambda g, i: (0, 0)),
            pl.BlockSpec((1, 1, 2 * HID), lambda g, i: (g, 0, 0)),
            pl.BlockSpec((2 * HID, HID), lambda g, i: (0, 0)),
            pl.BlockSpec((1, HID), lambda g, i: (0, 0)),
        ],
        out_specs=(
            pl.BlockSpec((TR, HID), lambda g, i: (g * nt + i, 0)),
            pl.BlockSpec((1, HID), lambda g, i: (0, 0)),
            pl.BlockSpec((1, HID), lambda g, i: (0, 0)),
        ),
        out_shape=(
            jax.ShapeDtypeStruct((G * NG, HID), jnp.float32),
            jax.ShapeDtypeStruct((1, HID), jnp.float32),
            jax.ShapeDtypeStruct((1, HID), jnp.float32),
        ),
    )(P, alpha, beta.reshape(G, 1, 2 * HID), W2t, b2)


def _k10_body(h2_ref, a_ref, c_ref, w_ref, b_ref, h3_ref, hs_ref, hq_ref):
    first = pl.program_id(0) == 0
    t = jnp.maximum(h2_ref[...] * a_ref[...] + c_ref[...], 0.0) * w_ref[...]
    row = jnp.sum(t, axis=1, keepdims=True) + b_ref[...]
    h3_ref[...] = row
    srow = jnp.sum(row, axis=0, keepdims=True)
    qrow = jnp.sum(row * row, axis=0, keepdims=True)

    @pl.when(first)
    def _():
        hs_ref[...] = srow
        hq_ref[...] = qrow

    @pl.when(jnp.logical_not(first))
    def _():
        hs_ref[...] = hs_ref[...] + srow
        hq_ref[...] = hq_ref[...] + qrow


def _tc_layer3(H2, a3, c3, w3, b3):
    nt = (G * NG) // TR
    return pl.pallas_call(
        _k10_body,
        grid=(nt,),
        in_specs=[
            pl.BlockSpec((TR, HID), lambda i: (i, 0)),
            pl.BlockSpec((1, HID), lambda i: (0, 0)),
            pl.BlockSpec((1, HID), lambda i: (0, 0)),
            pl.BlockSpec((1, HID), lambda i: (0, 0)),
            pl.BlockSpec((1, 1), lambda i: (0, 0)),
        ],
        out_specs=(
            pl.BlockSpec((TR, 1), lambda i: (i, 0)),
            pl.BlockSpec((1, 1), lambda i: (0, 0)),
            pl.BlockSpec((1, 1), lambda i: (0, 0)),
        ),
        out_shape=(
            jax.ShapeDtypeStruct((G * NG, 1), jnp.float32),
            jax.ShapeDtypeStruct((1, 1), jnp.float32),
            jax.ShapeDtypeStruct((1, 1), jnp.float32),
        ),
    )(H2, a3, c3, w3, b3)


def _k11_body(h3_ref, e_ref, a_ref, c_ref, out_ref):
    out_ref[...] = h3_ref[...] * a_ref[...] + c_ref[...] + e_ref[...]


def _tc_final(h3, expr, a4, c4):
    nt = (G * NG) // TR
    return pl.pallas_call(
        _k11_body,
        grid=(nt,),
        in_specs=[
            pl.BlockSpec((TR, 1), lambda i: (i, 0)),
            pl.BlockSpec((TR, 1), lambda i: (i, 0)),
            pl.BlockSpec((1, 1), lambda i: (0, 0)),
            pl.BlockSpec((1, 1), lambda i: (0, 0)),
        ],
        out_specs=pl.BlockSpec((TR, 1), lambda i: (i, 0)),
        out_shape=jax.ShapeDtypeStruct((G * NG, 1), jnp.float32),
    )(h3, expr, a4, c4)


def kernel(x, batch, G_sim, G_sim_weight, gene_emb, pert_emb,
           sg_W0, sg_b0, sg_W1, sg_b1,
           rec_W1, rec_b1, rec_W2, rec_b2, rec_W3, rec_b3):
    npad = EPAD - E
    src2 = jnp.concatenate([G_sim[0], jnp.zeros((npad,), G_sim.dtype)]).reshape(-1, ECH)
    dst2 = jnp.concatenate([G_sim[1], jnp.zeros((npad,), G_sim.dtype)]).reshape(-1, ECH)
    ew2 = jnp.concatenate(
        [G_sim_weight, jnp.zeros((npad,), G_sim_weight.dtype)]).reshape(-1, ECH)

    base, p0, bm, bsq = _tc_prep(gene_emb, pert_emb)

    degp = _sc_deg(dst2, ew2)
    dinv = _tc_dinv(degp)
    dinv_col = dinv.reshape(NGP)[:NG].reshape(NG, 1)

    y0 = _tc_rowscale(p0, dinv_col)
    s0 = _sc_edge_scatter(src2, dst2, ew2, y0)
    pge1 = _tc_sg_linear(s0, p0, dinv_col, sg_W0.T, sg_b0, do_relu=True)
    y1 = _tc_rowscale(pge1, dinv_col)
    s1 = _sc_edge_scatter(src2, dst2, ew2, y1)
    pge2 = _tc_sg_linear(s1, pge1, dinv_col, sg_W1.T, sg_b1, do_relu=False)

    pertmat = x[:, 1].reshape(G, NG)
    add = _tc_pert_mix(pertmat, pge2)

    am = jnp.mean(add, axis=0, keepdims=True)
    av = jnp.mean(add * add, axis=0, keepdims=True) - am * am
    m1 = bm + am
    v1 = (bsq - bm * bm) + av
    sd1 = jnp.sqrt(v1 + EPS)

    W1s = rec_W1.T / sd1.reshape(HID, 1)
    c1 = (m1 / sd1) @ rec_W1.T
    Q = (add / sd1) @ rec_W1.T + rec_b1
    P, Pm, Psq = _tc_layer1(base, W1s, c1)
    qm = jnp.mean(Q, axis=0, keepdims=True)
    qv = jnp.mean(Q * Q, axis=0, keepdims=True) - qm * qm
    m2 = Pm + qm
    sd2 = jnp.sqrt((Psq - Pm * Pm) + qv + EPS)
    alpha = 1.0 / sd2
    beta = (Q - m2) / sd2

    H2, Hs, Hq = _tc_layer2(P, alpha, beta, rec_W2.T, rec_b2.reshape(1, HID))
    m3 = Hs / (G * NG)
    v3 = Hq / (G * NG) - m3 * m3
    sd3 = jnp.sqrt(v3 + EPS)
    a3 = 1.0 / sd3
    c3 = -m3 / sd3

    h3, hs, hq = _tc_layer3(H2, a3, c3, rec_W3, rec_b3.reshape(1, 1))
    m4 = hs / (G * NG)
    v4 = hq / (G * NG) - m4 * m4
    sd4 = jnp.sqrt(v4 + EPS)

    return _tc_final(h3, x[:, 0:1], 1.0 / sd4, -m4 / sd4)


# speedup vs baseline: 7.8662x; 1.0020x over previous
import functools

import jax
import jax.numpy as jnp
from jax import lax
from jax.experimental import pallas as pl
from jax.experimental.pallas import tpu as pltpu
from jax.experimental.pallas import tpu_sc as plsc

NG = 10000
NGP = 10240
HID = 128
G = 8
E = 320000
CHUNK = 80
EPS = 1e-5
NORM_EPS = 1e-7
LAM = 0.2

_MESH = dict(core_axis_name="c", subcore_axis_name="s")


EPAD = 2560 * 128
ECH = 128
CPTD = 80
CPT = 160
CW = HID // 2
RPT = NGP // 16


def _sc_deg(dst2, ew2):
    @functools.partial(
        pl.kernel,
        out_type=jax.ShapeDtypeStruct((2, NGP), jnp.float32),
        mesh=plsc.VectorSubcoreMesh(**_MESH),
        scratch_types=[
            pltpu.VMEM_SHARED((NGP,), jnp.float32),
            pltpu.VMEM((CPTD, ECH), jnp.int32),
            pltpu.VMEM((CPTD, ECH), jnp.float32),
            pltpu.VMEM((640,), jnp.float32),
            pltpu.SemaphoreType.DMA,
        ],
    )
    def deg_kernel(dst_hbm, ew_hbm, out_hbm, acc_sh, dst_v, ew_v, zero_v, sem):
        c = lax.axis_index("c")
        s = lax.axis_index("s")
        wid = s * 2 + c

        @pl.loop(0, 40)
        def _fill(i):
            zero_v[pl.ds(i * 16, 16)] = jnp.zeros((16,), jnp.float32)

        pltpu.sync_copy(zero_v, acc_sh.at[pl.ds(s * 640, 640)])
        pltpu.sync_copy(dst_hbm.at[pl.ds(wid * CPTD, CPTD)], dst_v)
        pltpu.sync_copy(ew_hbm.at[pl.ds(wid * CPTD, CPTD)], ew_v)
        plsc.subcore_barrier()

        @pl.loop(0, CPTD // 8)
        def _groups(t):
            ds_ = [
                pltpu.async_copy(ew_v.at[t * 8 + u], acc_sh.at[dst_v.at[t * 8 + u]],
                                 sem, add=True)
                for u in range(8)
            ]
            for d in ds_:
                d.wait()

        plsc.subcore_barrier()
        pltpu.sync_copy(acc_sh.at[pl.ds(s * 640, 640)],
                        out_hbm.at[c, pl.ds(s * 640, 640)])

    return deg_kernel(dst2, ew2)


def _sc_edge_scatter(src2, dst2, ew2, table):
    @functools.partial(
        pl.kernel,
        out_type=jax.ShapeDtypeStruct((2, NGP, HID), jnp.float32),
        mesh=plsc.VectorSubcoreMesh(**_MESH),
        scratch_types=[
            pltpu.VMEM_SHARED((NGP, HID), jnp.float32),
            pltpu.VMEM((CPTD, ECH), jnp.int32),
            pltpu.VMEM((2, ECH), jnp.int32),
            pltpu.VMEM((2, ECH), jnp.float32),
            pltpu.VMEM((2, ECH, HID), jnp.float32),
            [pltpu.SemaphoreType.DMA] * 2,
            [pltpu.SemaphoreType.DMA] * 2,
            [pltpu.SemaphoreType.DMA] * 2,
        ],
    )
    def scat_kernel(src_hbm, dst_hbm, ew_hbm, tab_hbm, out_hbm,
                    acc_sh, src_v, dst_v, ew_v, rows_v, isems, gsems, ssems):
        c = lax.axis_index("c")
        s = lax.axis_index("s")
        wid = s * 2 + c
        base = wid * CPTD

        @pl.loop(0, ECH)
        def _z(r):
            for k in range(HID // 16):
                rows_v[0, r, pl.ds(k * 16, 16)] = jnp.zeros((16,), jnp.float32)

        for t in range(RPT // ECH):
            pltpu.sync_copy(rows_v.at[0], acc_sh.at[pl.ds(s * RPT + t * ECH, ECH)])
        pltpu.sync_copy(src_hbm.at[pl.ds(base, CPTD)], src_v)

        @pl.loop(0, CPTD)
        def _remap(r):
            @pl.loop(0, ECH // 16)
            def _rb(b):
                sl = pl.ds(b * 16, 16)
                src_v[r, sl] = src_v[r, sl] + c * NG

        plsc.subcore_barrier()

        def fire(j, u):
            pltpu.async_copy(dst_hbm.at[base + j], dst_v.at[u], isems[u])
            pltpu.async_copy(ew_hbm.at[base + j], ew_v.at[u], isems[u])
            pltpu.async_copy(tab_hbm.at[src_v.at[j]], rows_v.at[u], gsems[u])

        def wait_in(u):
            pltpu.make_async_copy(dst_hbm.at[0], dst_v.at[u], isems[u]).wait()
            pltpu.make_async_copy(ew_hbm.at[0], ew_v.at[u], isems[u]).wait()
            pltpu.make_async_copy(tab_hbm.at[src_v.at[0]], rows_v.at[u],
                                  gsems[u]).wait()

        def scale(u):
            @pl.loop(0, ECH // 16)
            def _scale(b):
                wv = ew_v[u, pl.ds(b * 16, 16)]
                for e in range(16):
                    w = wv[e]
                    for k in range(HID // 16):
                        sl = pl.ds(k * 16, 16)
                        rows_v[u, b * 16 + e, sl] = rows_v[u, b * 16 + e, sl] * w

        def scat(u):
            pltpu.async_copy(rows_v.at[u], acc_sh.at[dst_v.at[u]], ssems[u],
                             add=True)

        def wait_scat(u):
            pltpu.make_async_copy(rows_v.at[u], acc_sh.at[dst_v.at[u]],
                                  ssems[u]).wait()

        fire(0, 0)
        fire(1, 1)

        @pl.loop(0, CPTD // 2)
        def _pairs(t):
            j0 = t * 2
            wait_in(0)
            scale(0)
            scat(0)
            wait_in(1)
            scale(1)
            scat(1)
            wait_scat(0)

            @pl.when(j0 + 2 < CPTD)
            def _():
                fire(j0 + 2, 0)

            wait_scat(1)

            @pl.when(j0 + 3 < CPTD)
            def _():
                fire(j0 + 3, 1)

        plsc.subcore_barrier()
        pltpu.sync_copy(acc_sh.at[pl.ds(s * RPT, RPT)],
                        out_hbm.at[c, pl.ds(s * RPT, RPT)])

    return scat_kernel(src2, dst2, ew2, table)


def _k1_body(ge_ref, pe_ref, base_ref, p0_ref, bm_ref, bsq_ref):
    ge = ge_ref[...]
    n = jnp.sqrt(jnp.sum(ge * ge, axis=1, keepdims=True))
    r = ge * jnp.minimum(1.0, 1.0 / (n + NORM_EPS))
    m = jnp.mean(r, axis=0, keepdims=True)
    v = jnp.mean(r * r, axis=0, keepdims=True) - m * m
    base = jnp.maximum((r - m) / jnp.sqrt(v + EPS), 0.0)
    base_ref[...] = base
    bm_ref[...] = jnp.mean(base, axis=0, keepdims=True)
    bsq_ref[...] = jnp.mean(base * base, axis=0, keepdims=True)
    pe = pe_ref[...]
    n2 = jnp.sqrt(jnp.sum(pe * pe, axis=1, keepdims=True))
    p0_ref[...] = pe * jnp.minimum(1.0, 1.0 / (n2 + NORM_EPS))


def _tc_prep(gene_emb, pert_emb):
    return pl.pallas_call(
        _k1_body,
        out_shape=(
            jax.ShapeDtypeStruct((NG, HID), jnp.float32),
            jax.ShapeDtypeStruct((NG, HID), jnp.float32),
            jax.ShapeDtypeStruct((1, HID), jnp.float32),
            jax.ShapeDtypeStruct((1, HID), jnp.float32),
        ),
    )(gene_emb, pert_emb)


def _k5_body(sp_ref, xf_ref, dc_ref, w_ref, b_ref, out_ref, *, do_relu):
    dc = dc_ref[...]
    z = (sp_ref[0, :NG] + sp_ref[1, :NG]) * dc + xf_ref[...] * (dc * dc)
    h = jnp.dot(z, w_ref[...], preferred_element_type=jnp.float32) + b_ref[...]
    out_ref[...] = jnp.maximum(h, 0.0) if do_relu else h


def _tc_sg_linear(sp, xf, dinv_col, Wt, b, do_relu):
    return pl.pallas_call(
        functools.partial(_k5_body, do_relu=do_relu),
        out_shape=jax.ShapeDtypeStruct((NG, HID), jnp.float32),
    )(sp, xf, dinv_col, Wt, b.reshape(1, HID))


def _kdinv_body(dp_ref, out_ref):
    out_ref[...] = lax.rsqrt(dp_ref[0:1] + dp_ref[1:2] + 1.0)


def _tc_dinv(degp):
    return pl.pallas_call(
        _kdinv_body,
        out_shape=jax.ShapeDtypeStruct((1, NGP), jnp.float32),
    )(degp)


def _krs_body(t_ref, d_ref, out_ref):
    y = t_ref[...] * d_ref[...]
    out_ref[0] = y
    out_ref[1] = y


def _tc_rowscale(tab, dinv_col):
    return pl.pallas_call(
        _krs_body,
        out_shape=jax.ShapeDtypeStruct((2, NG, HID), jnp.float32),
    )(tab, dinv_col)


def _k7_body(pm_ref, pge_ref, add_ref):
    add_ref[...] = LAM * jnp.dot(pm_ref[...], pge_ref[...],
                                 preferred_element_type=jnp.float32)


def _tc_pert_mix(pertmat, pge2):
    return pl.pallas_call(
        _k7_body,
        out_shape=jax.ShapeDtypeStruct((G, HID), jnp.float32),
    )(pertmat, pge2)


def _k8_body(base_ref, w_ref, c_ref, p_ref, pm_ref, psq_ref):
    p = jnp.dot(base_ref[...], w_ref[...],
                preferred_element_type=jnp.float32) - c_ref[...]
    p_ref[...] = p
    pm_ref[...] = jnp.mean(p, axis=0, keepdims=True)
    psq_ref[...] = jnp.mean(p * p, axis=0, keepdims=True)


def _tc_layer1(base, W1s, c1):
    return pl.pallas_call(
        _k8_body,
        out_shape=(
            jax.ShapeDtypeStruct((NG, 2 * HID), jnp.float32),
            jax.ShapeDtypeStruct((1, 2 * HID), jnp.float32),
            jax.ShapeDtypeStruct((1, 2 * HID), jnp.float32),
        ),
    )(base, W1s, c1)


TR = 1000


def _k9_body(p_ref, al_ref, be_ref, w_ref, b_ref, h2_ref, hs_ref, hq_ref):
    first = jnp.logical_and(pl.program_id(0) == 0, pl.program_id(1) == 0)
    r = jnp.maximum(p_ref[...] * al_ref[...] + be_ref[0], 0.0)
    h2 = jnp.dot(r, w_ref[...], preferred_element_type=jnp.float32) + b_ref[...]
    h2_ref[...] = h2
    srow = jnp.sum(h2, axis=0, keepdims=True)
    qrow = jnp.sum(h2 * h2, axis=0, keepdims=True)

    @pl.when(first)
    def _():
        hs_ref[...] = srow
        hq_ref[...] = qrow

    @pl.when(jnp.logical_not(first))
    def _():
        hs_ref[...] = hs_ref[...] + srow
        hq_ref[...] = hq_ref[...] + qrow


def _tc_layer2(P, alpha, beta, W2t, b2):
    nt = NG // TR
    return pl.pallas_call(
        _k9_body,
        grid=(G, nt),
        in_specs=[
            pl.BlockSpec((TR, 2 * HID), lambda g, i: (i, 0)),
            pl.BlockSpec((1, 2 * HID), lambda g, i: (0, 0)),
            pl.BlockSpec((1, 1, 2 * HID), lambda g, i: (g, 0, 0)),
            pl.BlockSpec((2 * HID, HID), lambda g, i: (0, 0)),
            pl.BlockSpec((1, HID), lambda g, i: (0, 0)),
        ],
        out_specs=(
            pl.BlockSpec((TR, HID), lambda g, i: (g * nt + i, 0)),
            pl.BlockSpec((1, HID), lambda g, i: (0, 0)),
            pl.BlockSpec((1, HID), lambda g, i: (0, 0)),
        ),
        out_shape=(
            jax.ShapeDtypeStruct((G * NG, HID), jnp.float32),
            jax.ShapeDtypeStruct((1, HID), jnp.float32),
            jax.ShapeDtypeStruct((1, HID), jnp.float32),
        ),
    )(P, alpha, beta.reshape(G, 1, 2 * HID), W2t, b2)


def _k10_body(h2_ref, a_ref, c_ref, w_ref, b_ref, h3_ref, hs_ref, hq_ref):
    first = pl.program_id(0) == 0
    t = jnp.maximum(h2_ref[...] * a_ref[...] + c_ref[...], 0.0) * w_ref[...]
    row = jnp.sum(t, axis=1, keepdims=True) + b_ref[...]
    h3_ref[...] = row
    srow = jnp.sum(row, axis=0, keepdims=True)
    qrow = jnp.sum(row * row, axis=0, keepdims=True)

    @pl.when(first)
    def _():
        hs_ref[...] = srow
        hq_ref[...] = qrow

    @pl.when(jnp.logical_not(first))
    def _():
        hs_ref[...] = hs_ref[...] + srow
        hq_ref[...] = hq_ref[...] + qrow


def _tc_layer3(H2, a3, c3, w3, b3):
    nt = (G * NG) // TR
    return pl.pallas_call(
        _k10_body,
        grid=(nt,),
        in_specs=[
            pl.BlockSpec((TR, HID), lambda i: (i, 0)),
            pl.BlockSpec((1, HID), lambda i: (0, 0)),
            pl.BlockSpec((1, HID), lambda i: (0, 0)),
            pl.BlockSpec((1, HID), lambda i: (0, 0)),
            pl.BlockSpec((1, 1), lambda i: (0, 0)),
        ],
        out_specs=(
            pl.BlockSpec((TR, 1), lambda i: (i, 0)),
            pl.BlockSpec((1, 1), lambda i: (0, 0)),
            pl.BlockSpec((1, 1), lambda i: (0, 0)),
        ),
        out_shape=(
            jax.ShapeDtypeStruct((G * NG, 1), jnp.float32),
            jax.ShapeDtypeStruct((1, 1), jnp.float32),
            jax.ShapeDtypeStruct((1, 1), jnp.float32),
        ),
    )(H2, a3, c3, w3, b3)


def _k11_body(h3_ref, e_ref, a_ref, c_ref, out_ref):
    out_ref[...] = h3_ref[...] * a_ref[...] + c_ref[...] + e_ref[...]


def _tc_final(h3, expr, a4, c4):
    nt = (G * NG) // TR
    return pl.pallas_call(
        _k11_body,
        grid=(nt,),
        in_specs=[
            pl.BlockSpec((TR, 1), lambda i: (i, 0)),
            pl.BlockSpec((TR, 1), lambda i: (i, 0)),
            pl.BlockSpec((1, 1), lambda i: (0, 0)),
            pl.BlockSpec((1, 1), lambda i: (0, 0)),
        ],
        out_specs=pl.BlockSpec((TR, 1), lambda i: (i, 0)),
        out_shape=jax.ShapeDtypeStruct((G * NG, 1), jnp.float32),
    )(h3, expr, a4, c4)


def kernel(x, batch, G_sim, G_sim_weight, gene_emb, pert_emb,
           sg_W0, sg_b0, sg_W1, sg_b1,
           rec_W1, rec_b1, rec_W2, rec_b2, rec_W3, rec_b3):
    npad = EPAD - E
    src2 = jnp.concatenate([G_sim[0], jnp.zeros((npad,), G_sim.dtype)]).reshape(-1, ECH)
    dst2 = jnp.concatenate([G_sim[1], jnp.zeros((npad,), G_sim.dtype)]).reshape(-1, ECH)
    ew2 = jnp.concatenate(
        [G_sim_weight, jnp.zeros((npad,), G_sim_weight.dtype)]).reshape(-1, ECH)

    base, p0, bm, bsq = _tc_prep(gene_emb, pert_emb)

    degp = _sc_deg(dst2, ew2)
    dinv = _tc_dinv(degp)
    dinv_col = dinv.reshape(NGP)[:NG].reshape(NG, 1)

    y0 = _tc_rowscale(p0, dinv_col)
    s0 = _sc_edge_scatter(src2, dst2, ew2, y0.reshape(2 * NG, HID))
    pge1 = _tc_sg_linear(s0, p0, dinv_col, sg_W0.T, sg_b0, do_relu=True)
    y1 = _tc_rowscale(pge1, dinv_col)
    s1 = _sc_edge_scatter(src2, dst2, ew2, y1.reshape(2 * NG, HID))
    pge2 = _tc_sg_linear(s1, pge1, dinv_col, sg_W1.T, sg_b1, do_relu=False)

    pertmat = x[:, 1].reshape(G, NG)
    add = _tc_pert_mix(pertmat, pge2)

    am = jnp.mean(add, axis=0, keepdims=True)
    av = jnp.mean(add * add, axis=0, keepdims=True) - am * am
    m1 = bm + am
    v1 = (bsq - bm * bm) + av
    sd1 = jnp.sqrt(v1 + EPS)

    W1s = rec_W1.T / sd1.reshape(HID, 1)
    c1 = (m1 / sd1) @ rec_W1.T
    Q = (add / sd1) @ rec_W1.T + rec_b1
    P, Pm, Psq = _tc_layer1(base, W1s, c1)
    qm = jnp.mean(Q, axis=0, keepdims=True)
    qv = jnp.mean(Q * Q, axis=0, keepdims=True) - qm * qm
    m2 = Pm + qm
    sd2 = jnp.sqrt((Psq - Pm * Pm) + qv + EPS)
    alpha = 1.0 / sd2
    beta = (Q - m2) / sd2

    H2, Hs, Hq = _tc_layer2(P, alpha, beta, rec_W2.T, rec_b2.reshape(1, HID))
    m3 = Hs / (G * NG)
    v3 = Hq / (G * NG) - m3 * m3
    sd3 = jnp.sqrt(v3 + EPS)
    a3 = 1.0 / sd3
    c3 = -m3 / sd3

    h3, hs, hq = _tc_layer3(H2, a3, c3, rec_W3, rec_b3.reshape(1, 1))
    m4 = hs / (G * NG)
    v4 = hq / (G * NG) - m4 * m4
    sd4 = jnp.sqrt(v4 + EPS)

    return _tc_final(h3, x[:, 0:1], 1.0 / sd4, -m4 / sd4)


# speedup vs baseline: 8.6359x; 1.0978x over previous
import functools

import jax
import jax.numpy as jnp
from jax import lax
from jax.experimental import pallas as pl
from jax.experimental.pallas import tpu as pltpu
from jax.experimental.pallas import tpu_sc as plsc

NG = 10000
NGP = 10240
HID = 128
G = 8
E = 320000
CHUNK = 80
EPS = 1e-5
NORM_EPS = 1e-7
LAM = 0.2

_MESH = dict(core_axis_name="c", subcore_axis_name="s")


EPAD = 2560 * 128
ECH = 128
CPTD = 80
CPT = 160
CW = HID // 2
RPT = NGP // 16
CF = 120
CS = 40


def _sc_deg(dst2, ew2):
    @functools.partial(
        pl.kernel,
        out_type=jax.ShapeDtypeStruct((2, NGP), jnp.float32),
        mesh=plsc.VectorSubcoreMesh(**_MESH),
        scratch_types=[
            pltpu.VMEM_SHARED((NGP,), jnp.float32),
            pltpu.VMEM((CPTD, ECH), jnp.int32),
            pltpu.VMEM((CPTD, ECH), jnp.float32),
            pltpu.VMEM((640,), jnp.float32),
            pltpu.SemaphoreType.DMA,
        ],
    )
    def deg_kernel(dst_hbm, ew_hbm, out_hbm, acc_sh, dst_v, ew_v, zero_v, sem):
        c = lax.axis_index("c")
        s = lax.axis_index("s")
        wid = s * 2 + c

        @pl.loop(0, 40)
        def _fill(i):
            zero_v[pl.ds(i * 16, 16)] = jnp.zeros((16,), jnp.float32)

        pltpu.sync_copy(zero_v, acc_sh.at[pl.ds(s * 640, 640)])
        pltpu.sync_copy(dst_hbm.at[pl.ds(wid * CPTD, CPTD)], dst_v)
        pltpu.sync_copy(ew_hbm.at[pl.ds(wid * CPTD, CPTD)], ew_v)
        plsc.subcore_barrier()

        @pl.loop(0, CPTD // 8)
        def _groups(t):
            ds_ = [
                pltpu.async_copy(ew_v.at[t * 8 + u], acc_sh.at[dst_v.at[t * 8 + u]],
                                 sem, add=True)
                for u in range(8)
            ]
            for d in ds_:
                d.wait()

        plsc.subcore_barrier()
        pltpu.sync_copy(acc_sh.at[pl.ds(s * 640, 640)],
                        out_hbm.at[c, pl.ds(s * 640, 640)])

    return deg_kernel(dst2, ew2)


def _sc_edge_scatter(src2, dst2, ew2, table):
    @functools.partial(
        pl.kernel,
        out_type=jax.ShapeDtypeStruct((2, NGP, HID), jnp.float32),
        mesh=plsc.VectorSubcoreMesh(**_MESH),
        scratch_types=[
            pltpu.VMEM_SHARED((NGP, HID), jnp.float32),
            pltpu.VMEM((CF, ECH), jnp.int32),
            pltpu.VMEM((2, ECH), jnp.int32),
            pltpu.VMEM((2, ECH), jnp.float32),
            pltpu.VMEM((2, ECH, HID), jnp.float32),
            [pltpu.SemaphoreType.DMA] * 2,
            [pltpu.SemaphoreType.DMA] * 2,
            [pltpu.SemaphoreType.DMA] * 2,
        ],
    )
    def scat_kernel(src_hbm, dst_hbm, ew_hbm, tab_hbm, out_hbm,
                    acc_sh, src_v, dst_v, ew_v, rows_v, isems, gsems, ssems):
        c = lax.axis_index("c")
        s = lax.axis_index("s")
        base = jnp.where(c == 0, s * CF, 16 * CF + s * CS)
        nc = jnp.where(c == 0, CF, CS)

        @pl.loop(0, ECH)
        def _z(r):
            for k in range(HID // 16):
                rows_v[0, r, pl.ds(k * 16, 16)] = jnp.zeros((16,), jnp.float32)

        for t in range(RPT // ECH):
            pltpu.sync_copy(rows_v.at[0], acc_sh.at[pl.ds(s * RPT + t * ECH, ECH)])
        pltpu.sync_copy(src_hbm.at[pl.ds(base, CS)], src_v.at[pl.ds(0, CS)])

        @pl.when(c == 0)
        def _more():
            pltpu.sync_copy(src_hbm.at[pl.ds(base + CS, CF - CS)],
                            src_v.at[pl.ds(CS, CF - CS)])

        @pl.loop(0, CF)
        def _remap(r):
            @pl.loop(0, ECH // 16)
            def _rb(b):
                sl = pl.ds(b * 16, 16)
                src_v[r, sl] = src_v[r, sl] + c * NG

        plsc.subcore_barrier()

        def fire(j, u):
            pltpu.async_copy(dst_hbm.at[base + j], dst_v.at[u], isems[u])
            pltpu.async_copy(ew_hbm.at[base + j], ew_v.at[u], isems[u])
            pltpu.async_copy(tab_hbm.at[src_v.at[j]], rows_v.at[u], gsems[u])

        def wait_in(u):
            pltpu.make_async_copy(dst_hbm.at[0], dst_v.at[u], isems[u]).wait()
            pltpu.make_async_copy(ew_hbm.at[0], ew_v.at[u], isems[u]).wait()
            pltpu.make_async_copy(tab_hbm.at[src_v.at[0]], rows_v.at[u],
                                  gsems[u]).wait()

        def scale(u):
            @pl.loop(0, ECH // 16)
            def _scale(b):
                wv = ew_v[u, pl.ds(b * 16, 16)]
                for e in range(16):
                    w = wv[e]
                    for k in range(HID // 16):
                        sl = pl.ds(k * 16, 16)
                        rows_v[u, b * 16 + e, sl] = rows_v[u, b * 16 + e, sl] * w

        def scat(u):
            pltpu.async_copy(rows_v.at[u], acc_sh.at[dst_v.at[u]], ssems[u],
                             add=True)

        def wait_scat(u):
            pltpu.make_async_copy(rows_v.at[u], acc_sh.at[dst_v.at[u]],
                                  ssems[u]).wait()

        fire(0, 0)
        fire(1, 1)

        @pl.loop(0, jnp.where(c == 0, CF // 2, CS // 2))
        def _pairs(t):
            j0 = t * 2
            wait_in(0)
            scale(0)
            scat(0)
            wait_in(1)
            scale(1)
            scat(1)
            wait_scat(0)

            @pl.when(j0 + 2 < nc)
            def _():
                fire(j0 + 2, 0)

            wait_scat(1)

            @pl.when(j0 + 3 < nc)
            def _():
                fire(j0 + 3, 1)

        plsc.subcore_barrier()
        pltpu.sync_copy(acc_sh.at[pl.ds(s * RPT, RPT)],
                        out_hbm.at[c, pl.ds(s * RPT, RPT)])

    return scat_kernel(src2, dst2, ew2, table)


def _k1_body(ge_ref, pe_ref, base_ref, p0_ref, bm_ref, bsq_ref):
    ge = ge_ref[...]
    n = jnp.sqrt(jnp.sum(ge * ge, axis=1, keepdims=True))
    r = ge * jnp.minimum(1.0, 1.0 / (n + NORM_EPS))
    m = jnp.mean(r, axis=0, keepdims=True)
    v = jnp.mean(r * r, axis=0, keepdims=True) - m * m
    base = jnp.maximum((r - m) / jnp.sqrt(v + EPS), 0.0)
    base_ref[...] = base
    bm_ref[...] = jnp.mean(base, axis=0, keepdims=True)
    bsq_ref[...] = jnp.mean(base * base, axis=0, keepdims=True)
    pe = pe_ref[...]
    n2 = jnp.sqrt(jnp.sum(pe * pe, axis=1, keepdims=True))
    p0_ref[...] = pe * jnp.minimum(1.0, 1.0 / (n2 + NORM_EPS))


def _tc_prep(gene_emb, pert_emb):
    return pl.pallas_call(
        _k1_body,
        out_shape=(
            jax.ShapeDtypeStruct((NG, HID), jnp.float32),
            jax.ShapeDtypeStruct((NG, HID), jnp.float32),
            jax.ShapeDtypeStruct((1, HID), jnp.float32),
            jax.ShapeDtypeStruct((1, HID), jnp.float32),
        ),
    )(gene_emb, pert_emb)


def _k5_body(sp_ref, xf_ref, dc_ref, w_ref, b_ref, out_ref, *, do_relu):
    dc = dc_ref[...]
    z = (sp_ref[0, :NG] + sp_ref[1, :NG]) * dc + xf_ref[...] * (dc * dc)
    h = jnp.dot(z, w_ref[...], preferred_element_type=jnp.float32) + b_ref[...]
    out_ref[...] = jnp.maximum(h, 0.0) if do_relu else h


def _tc_sg_linear(sp, xf, dinv_col, Wt, b, do_relu):
    return pl.pallas_call(
        functools.partial(_k5_body, do_relu=do_relu),
        out_shape=jax.ShapeDtypeStruct((NG, HID), jnp.float32),
    )(sp, xf, dinv_col, Wt, b.reshape(1, HID))


def _kdinv_body(dp_ref, out_ref):
    out_ref[...] = lax.rsqrt(dp_ref[0:1] + dp_ref[1:2] + 1.0)


def _tc_dinv(degp):
    return pl.pallas_call(
        _kdinv_body,
        out_shape=jax.ShapeDtypeStruct((1, NGP), jnp.float32),
    )(degp)


def _krs_body(t_ref, d_ref, out_ref):
    y = t_ref[...] * d_ref[...]
    out_ref[0] = y
    out_ref[1] = y


def _tc_rowscale(tab, dinv_col):
    return pl.pallas_call(
        _krs_body,
        out_shape=jax.ShapeDtypeStruct((2, NG, HID), jnp.float32),
    )(tab, dinv_col)


def _k7_body(pm_ref, pge_ref, add_ref):
    add_ref[...] = LAM * jnp.dot(pm_ref[...], pge_ref[...],
                                 preferred_element_type=jnp.float32)


def _tc_pert_mix(pertmat, pge2):
    return pl.pallas_call(
        _k7_body,
        out_shape=jax.ShapeDtypeStruct((G, HID), jnp.float32),
    )(pertmat, pge2)


def _k8_body(base_ref, w_ref, c_ref, p_ref, pm_ref, psq_ref):
    p = jnp.dot(base_ref[...], w_ref[...],
                preferred_element_type=jnp.float32) - c_ref[...]
    p_ref[...] = p
    pm_ref[...] = jnp.mean(p, axis=0, keepdims=True)
    psq_ref[...] = jnp.mean(p * p, axis=0, keepdims=True)


def _tc_layer1(base, W1s, c1):
    return pl.pallas_call(
        _k8_body,
        out_shape=(
            jax.ShapeDtypeStruct((NG, 2 * HID), jnp.float32),
            jax.ShapeDtypeStruct((1, 2 * HID), jnp.float32),
            jax.ShapeDtypeStruct((1, 2 * HID), jnp.float32),
        ),
    )(base, W1s, c1)


TR = 1000


def _k9_body(p_ref, al_ref, be_ref, w_ref, b_ref, h2_ref, hs_ref, hq_ref):
    first = jnp.logical_and(pl.program_id(0) == 0, pl.program_id(1) == 0)
    r = jnp.maximum(p_ref[...] * al_ref[...] + be_ref[0], 0.0)
    h2 = jnp.dot(r, w_ref[...], preferred_element_type=jnp.float32) + b_ref[...]
    h2_ref[...] = h2
    srow = jnp.sum(h2, axis=0, keepdims=True)
    qrow = jnp.sum(h2 * h2, axis=0, keepdims=True)

    @pl.when(first)
    def _():
        hs_ref[...] = srow
        hq_ref[...] = qrow

    @pl.when(jnp.logical_not(first))
    def _():
        hs_ref[...] = hs_ref[...] + srow
        hq_ref[...] = hq_ref[...] + qrow


def _tc_layer2(P, alpha, beta, W2t, b2):
    nt = NG // TR
    return pl.pallas_call(
        _k9_body,
        grid=(G, nt),
        in_specs=[
            pl.BlockSpec((TR, 2 * HID), lambda g, i: (i, 0)),
            pl.BlockSpec((1, 2 * HID), lambda g, i: (0, 0)),
            pl.BlockSpec((1, 1, 2 * HID), lambda g, i: (g, 0, 0)),
            pl.BlockSpec((2 * HID, HID), lambda g, i: (0, 0)),
            pl.BlockSpec((1, HID), lambda g, i: (0, 0)),
        ],
        out_specs=(
            pl.BlockSpec((TR, HID), lambda g, i: (g * nt + i, 0)),
            pl.BlockSpec((1, HID), lambda g, i: (0, 0)),
            pl.BlockSpec((1, HID), lambda g, i: (0, 0)),
        ),
        out_shape=(
            jax.ShapeDtypeStruct((G * NG, HID), jnp.float32),
            jax.ShapeDtypeStruct((1, HID), jnp.float32),
            jax.ShapeDtypeStruct((1, HID), jnp.float32),
        ),
    )(P, alpha, beta.reshape(G, 1, 2 * HID), W2t, b2)


def _k10_body(h2_ref, a_ref, c_ref, w_ref, b_ref, h3_ref, hs_ref, hq_ref):
    first = pl.program_id(0) == 0
    t = jnp.maximum(h2_ref[...] * a_ref[...] + c_ref[...], 0.0) * w_ref[...]
    row = jnp.sum(t, axis=1, keepdims=True) + b_ref[...]
    h3_ref[...] = row
    srow = jnp.sum(row, axis=0, keepdims=True)
    qrow = jnp.sum(row * row, axis=0, keepdims=True)

    @pl.when(first)
    def _():
        hs_ref[...] = srow
        hq_ref[...] = qrow

    @pl.when(jnp.logical_not(first))
    def _():
        hs_ref[...] = hs_ref[...] + srow
        hq_ref[...] = hq_ref[...] + qrow


def _tc_layer3(H2, a3, c3, w3, b3):
    nt = (G * NG) // TR
    return pl.pallas_call(
        _k10_body,
        grid=(nt,),
        in_specs=[
            pl.BlockSpec((TR, HID), lambda i: (i, 0)),
            pl.BlockSpec((1, HID), lambda i: (0, 0)),
            pl.BlockSpec((1, HID), lambda i: (0, 0)),
            pl.BlockSpec((1, HID), lambda i: (0, 0)),
            pl.BlockSpec((1, 1), lambda i: (0, 0)),
        ],
        out_specs=(
            pl.BlockSpec((TR, 1), lambda i: (i, 0)),
            pl.BlockSpec((1, 1), lambda i: (0, 0)),
            pl.BlockSpec((1, 1), lambda i: (0, 0)),
        ),
        out_shape=(
            jax.ShapeDtypeStruct((G * NG, 1), jnp.float32),
            jax.ShapeDtypeStruct((1, 1), jnp.float32),
            jax.ShapeDtypeStruct((1, 1), jnp.float32),
        ),
    )(H2, a3, c3, w3, b3)


def _k11_body(h3_ref, e_ref, a_ref, c_ref, out_ref):
    out_ref[...] = h3_ref[...] * a_ref[...] + c_ref[...] + e_ref[...]


def _tc_final(h3, expr, a4, c4):
    nt = (G * NG) // TR
    return pl.pallas_call(
        _k11_body,
        grid=(nt,),
        in_specs=[
            pl.BlockSpec((TR, 1), lambda i: (i, 0)),
            pl.BlockSpec((TR, 1), lambda i: (i, 0)),
            pl.BlockSpec((1, 1), lambda i: (0, 0)),
            pl.BlockSpec((1, 1), lambda i: (0, 0)),
        ],
        out_specs=pl.BlockSpec((TR, 1), lambda i: (i, 0)),
        out_shape=jax.ShapeDtypeStruct((G * NG, 1), jnp.float32),
    )(h3, expr, a4, c4)


def kernel(x, batch, G_sim, G_sim_weight, gene_emb, pert_emb,
           sg_W0, sg_b0, sg_W1, sg_b1,
           rec_W1, rec_b1, rec_W2, rec_b2, rec_W3, rec_b3):
    npad = EPAD - E
    src2 = jnp.concatenate([G_sim[0], jnp.zeros((npad,), G_sim.dtype)]).reshape(-1, ECH)
    dst2 = jnp.concatenate([G_sim[1], jnp.zeros((npad,), G_sim.dtype)]).reshape(-1, ECH)
    ew2 = jnp.concatenate(
        [G_sim_weight, jnp.zeros((npad,), G_sim_weight.dtype)]).reshape(-1, ECH)

    base, p0, bm, bsq = _tc_prep(gene_emb, pert_emb)

    degp = _sc_deg(dst2, ew2)
    dinv = _tc_dinv(degp)
    dinv_col = dinv.reshape(NGP)[:NG].reshape(NG, 1)

    y0 = _tc_rowscale(p0, dinv_col)
    s0 = _sc_edge_scatter(src2, dst2, ew2, y0.reshape(2 * NG, HID))
    pge1 = _tc_sg_linear(s0, p0, dinv_col, sg_W0.T, sg_b0, do_relu=True)
    y1 = _tc_rowscale(pge1, dinv_col)
    s1 = _sc_edge_scatter(src2, dst2, ew2, y1.reshape(2 * NG, HID))
    pge2 = _tc_sg_linear(s1, pge1, dinv_col, sg_W1.T, sg_b1, do_relu=False)

    pertmat = x[:, 1].reshape(G, NG)
    add = _tc_pert_mix(pertmat, pge2)

    am = jnp.mean(add, axis=0, keepdims=True)
    av = jnp.mean(add * add, axis=0, keepdims=True) - am * am
    m1 = bm + am
    v1 = (bsq - bm * bm) + av
    sd1 = jnp.sqrt(v1 + EPS)

    W1s = rec_W1.T / sd1.reshape(HID, 1)
    c1 = (m1 / sd1) @ rec_W1.T
    Q = (add / sd1) @ rec_W1.T + rec_b1
    P, Pm, Psq = _tc_layer1(base, W1s, c1)
    qm = jnp.mean(Q, axis=0, keepdims=True)
    qv = jnp.mean(Q * Q, axis=0, keepdims=True) - qm * qm
    m2 = Pm + qm
    sd2 = jnp.sqrt((Psq - Pm * Pm) + qv + EPS)
    alpha = 1.0 / sd2
    beta = (Q - m2) / sd2

    H2, Hs, Hq = _tc_layer2(P, alpha, beta, rec_W2.T, rec_b2.reshape(1, HID))
    m3 = Hs / (G * NG)
    v3 = Hq / (G * NG) - m3 * m3
    sd3 = jnp.sqrt(v3 + EPS)
    a3 = 1.0 / sd3
    c3 = -m3 / sd3

    h3, hs, hq = _tc_layer3(H2, a3, c3, rec_W3, rec_b3.reshape(1, 1))
    m4 = hs / (G * NG)
    v4 = hq / (G * NG) - m4 * m4
    sd4 = jnp.sqrt(v4 + EPS)

    return _tc_final(h3, x[:, 0:1], 1.0 / sd4, -m4 / sd4)


# speedup vs baseline: 9.0553x; 1.0486x over previous
import functools

import jax
import jax.numpy as jnp
from jax import lax
from jax.experimental import pallas as pl
from jax.experimental.pallas import tpu as pltpu
from jax.experimental.pallas import tpu_sc as plsc

NG = 10000
NGP = 10240
HID = 128
G = 8
E = 320000
CHUNK = 80
EPS = 1e-5
NORM_EPS = 1e-7
LAM = 0.2

_MESH = dict(core_axis_name="c", subcore_axis_name="s")


EPAD = 2560 * 128
ECH = 128
CPTD = 80
CPT = 160
CW = HID // 2
RPT = NGP // 16
CF = 120
CS = 40


def _sc_deg(dst2, ew2):
    @functools.partial(
        pl.kernel,
        out_type=jax.ShapeDtypeStruct((2, NGP), jnp.float32),
        mesh=plsc.VectorSubcoreMesh(**_MESH),
        scratch_types=[
            pltpu.VMEM_SHARED((NGP,), jnp.float32),
            pltpu.VMEM((CPTD, ECH), jnp.int32),
            pltpu.VMEM((CPTD, ECH), jnp.float32),
            pltpu.VMEM((640,), jnp.float32),
            pltpu.SemaphoreType.DMA,
        ],
    )
    def deg_kernel(dst_hbm, ew_hbm, out_hbm, acc_sh, dst_v, ew_v, zero_v, sem):
        c = lax.axis_index("c")
        s = lax.axis_index("s")
        wid = s * 2 + c

        @pl.loop(0, 40)
        def _fill(i):
            zero_v[pl.ds(i * 16, 16)] = jnp.zeros((16,), jnp.float32)

        pltpu.sync_copy(zero_v, acc_sh.at[pl.ds(s * 640, 640)])
        pltpu.sync_copy(dst_hbm.at[pl.ds(wid * CPTD, CPTD)], dst_v)
        pltpu.sync_copy(ew_hbm.at[pl.ds(wid * CPTD, CPTD)], ew_v)
        plsc.subcore_barrier()

        @pl.loop(0, CPTD // 8)
        def _groups(t):
            ds_ = [
                pltpu.async_copy(ew_v.at[t * 8 + u], acc_sh.at[dst_v.at[t * 8 + u]],
                                 sem, add=True)
                for u in range(8)
            ]
            for d in ds_:
                d.wait()

        plsc.subcore_barrier()
        pltpu.sync_copy(acc_sh.at[pl.ds(s * 640, 640)],
                        out_hbm.at[c, pl.ds(s * 640, 640)])

    return deg_kernel(dst2, ew2)


def _sc_edge_scatter(src2, dst2, ew2, table):
    @functools.partial(
        pl.kernel,
        out_type=jax.ShapeDtypeStruct((2, NGP, HID), jnp.float32),
        mesh=plsc.VectorSubcoreMesh(**_MESH),
        scratch_types=[
            pltpu.VMEM_SHARED((NGP, HID), jnp.float32),
            pltpu.VMEM((CF, ECH), jnp.int32),
            pltpu.VMEM((2, ECH), jnp.int32),
            pltpu.VMEM((2, ECH), jnp.float32),
            pltpu.VMEM((2, ECH, HID), jnp.float32),
            [pltpu.SemaphoreType.DMA] * 2,
            [pltpu.SemaphoreType.DMA] * 2,
            [pltpu.SemaphoreType.DMA] * 2,
        ],
    )
    def scat_kernel(src_hbm, dst_hbm, ew_hbm, tab_hbm, out_hbm,
                    acc_sh, src_v, dst_v, ew_v, rows_v, isems, gsems, ssems):
        c = lax.axis_index("c")
        s = lax.axis_index("s")
        base = jnp.where(c == 0, s * CF, 16 * CF + s * CS)
        nc = jnp.where(c == 0, CF, CS)

        @pl.loop(0, ECH)
        def _z(r):
            for k in range(HID // 16):
                rows_v[0, r, pl.ds(k * 16, 16)] = jnp.zeros((16,), jnp.float32)

        for t in range(RPT // ECH):
            pltpu.sync_copy(rows_v.at[0], acc_sh.at[pl.ds(s * RPT + t * ECH, ECH)])
        pltpu.sync_copy(src_hbm.at[pl.ds(base, CS)], src_v.at[pl.ds(0, CS)])

        @pl.when(c == 0)
        def _more():
            pltpu.sync_copy(src_hbm.at[pl.ds(base + CS, CF - CS)],
                            src_v.at[pl.ds(CS, CF - CS)])

        @pl.loop(0, CF)
        def _remap(r):
            @pl.loop(0, ECH // 16)
            def _rb(b):
                sl = pl.ds(b * 16, 16)
                src_v[r, sl] = src_v[r, sl] + c * NG

        plsc.subcore_barrier()

        def fire(j, u):
            pltpu.async_copy(dst_hbm.at[base + j], dst_v.at[u], isems[u])
            pltpu.async_copy(ew_hbm.at[base + j], ew_v.at[u], isems[u])
            pltpu.async_copy(tab_hbm.at[src_v.at[j]], rows_v.at[u], gsems[u])

        def wait_in(u):
            pltpu.make_async_copy(dst_hbm.at[0], dst_v.at[u], isems[u]).wait()
            pltpu.make_async_copy(ew_hbm.at[0], ew_v.at[u], isems[u]).wait()
            pltpu.make_async_copy(tab_hbm.at[src_v.at[0]], rows_v.at[u],
                                  gsems[u]).wait()

        def scale(u):
            @pl.loop(0, ECH // 16)
            def _scale(b):
                wv = ew_v[u, pl.ds(b * 16, 16)]
                for e in range(16):
                    w = wv[e]
                    for k in range(HID // 16):
                        sl = pl.ds(k * 16, 16)
                        rows_v[u, b * 16 + e, sl] = rows_v[u, b * 16 + e, sl] * w

        def scat(u):
            pltpu.async_copy(rows_v.at[u], acc_sh.at[dst_v.at[u]], ssems[u],
                             add=True)

        def wait_scat(u):
            pltpu.make_async_copy(rows_v.at[u], acc_sh.at[dst_v.at[u]],
                                  ssems[u]).wait()

        fire(0, 0)
        fire(1, 1)

        @pl.loop(0, jnp.where(c == 0, CF // 2, CS // 2))
        def _pairs(t):
            j0 = t * 2
            wait_in(0)
            scale(0)
            scat(0)
            wait_in(1)
            scale(1)
            scat(1)
            wait_scat(0)

            @pl.when(j0 + 2 < nc)
            def _():
                fire(j0 + 2, 0)

            wait_scat(1)

            @pl.when(j0 + 3 < nc)
            def _():
                fire(j0 + 3, 1)

        plsc.subcore_barrier()
        pltpu.sync_copy(acc_sh.at[pl.ds(s * RPT, RPT)],
                        out_hbm.at[c, pl.ds(s * RPT, RPT)])

    return scat_kernel(src2, dst2, ew2, table)


def _k1_body(ge_ref, pe_ref, base_ref, p0_ref, bm_ref, bsq_ref):
    ge = ge_ref[...]
    n = jnp.sqrt(jnp.sum(ge * ge, axis=1, keepdims=True))
    r = ge * jnp.minimum(1.0, 1.0 / (n + NORM_EPS))
    m = jnp.mean(r, axis=0, keepdims=True)
    v = jnp.mean(r * r, axis=0, keepdims=True) - m * m
    base = jnp.maximum((r - m) / jnp.sqrt(v + EPS), 0.0)
    base_ref[...] = base
    bm_ref[...] = jnp.mean(base, axis=0, keepdims=True)
    bsq_ref[...] = jnp.mean(base * base, axis=0, keepdims=True)
    pe = pe_ref[...]
    n2 = jnp.sqrt(jnp.sum(pe * pe, axis=1, keepdims=True))
    p0_ref[...] = pe * jnp.minimum(1.0, 1.0 / (n2 + NORM_EPS))


def _tc_prep(gene_emb, pert_emb):
    return pl.pallas_call(
        _k1_body,
        out_shape=(
            jax.ShapeDtypeStruct((NG, HID), jnp.float32),
            jax.ShapeDtypeStruct((NG, HID), jnp.float32),
            jax.ShapeDtypeStruct((1, HID), jnp.float32),
            jax.ShapeDtypeStruct((1, HID), jnp.float32),
        ),
    )(gene_emb, pert_emb)


def _k5_body(sp_ref, xf_ref, dc_ref, w_ref, b_ref, out_ref, *, do_relu):
    dc = dc_ref[...]
    z = (sp_ref[0, :NG] + sp_ref[1, :NG]) * dc + xf_ref[...] * (dc * dc)
    h = jnp.dot(z, w_ref[...], preferred_element_type=jnp.float32) + b_ref[...]
    out_ref[...] = jnp.maximum(h, 0.0) if do_relu else h


def _tc_sg_linear(sp, xf, dinv_col, Wt, b, do_relu):
    return pl.pallas_call(
        functools.partial(_k5_body, do_relu=do_relu),
        out_shape=jax.ShapeDtypeStruct((NG, HID), jnp.float32),
    )(sp, xf, dinv_col, Wt, b.reshape(1, HID))


def _kdinv_body(dp_ref, out_ref):
    out_ref[...] = lax.rsqrt(dp_ref[0:1] + dp_ref[1:2] + 1.0)


def _tc_dinv(degp):
    return pl.pallas_call(
        _kdinv_body,
        out_shape=jax.ShapeDtypeStruct((1, NGP), jnp.float32),
    )(degp)


def _krs_body(t_ref, d_ref, out_ref):
    y = t_ref[...] * d_ref[...]
    out_ref[0] = y
    out_ref[1] = y


def _tc_rowscale(tab, dinv_col):
    return pl.pallas_call(
        _krs_body,
        out_shape=jax.ShapeDtypeStruct((2, NG, HID), jnp.float32),
    )(tab, dinv_col)


def _k7_body(pm_ref, pge_ref, add_ref):
    add_ref[...] = LAM * jnp.dot(pm_ref[...], pge_ref[...],
                                 preferred_element_type=jnp.float32)


def _tc_pert_mix(pertmat, pge2):
    return pl.pallas_call(
        _k7_body,
        out_shape=jax.ShapeDtypeStruct((G, HID), jnp.float32),
    )(pertmat, pge2)


def _k8_body(base_ref, w_ref, c_ref, p_ref, pm_ref, psq_ref):
    p = jnp.dot(base_ref[...], w_ref[...],
                preferred_element_type=jnp.float32) - c_ref[...]
    p_ref[...] = p
    pm_ref[...] = jnp.mean(p, axis=0, keepdims=True)
    psq_ref[...] = jnp.mean(p * p, axis=0, keepdims=True)


def _tc_layer1(base, W1s, c1):
    return pl.pallas_call(
        _k8_body,
        out_shape=(
            jax.ShapeDtypeStruct((NG, 2 * HID), jnp.float32),
            jax.ShapeDtypeStruct((1, 2 * HID), jnp.float32),
            jax.ShapeDtypeStruct((1, 2 * HID), jnp.float32),
        ),
    )(base, W1s, c1)


TR = 1000


def _k9_body(p_ref, al_ref, be_ref, w_ref, b_ref, h2_ref, hs_ref, hq_ref):
    first = pl.program_id(0) == 0
    pa = p_ref[...] * al_ref[...]
    w = w_ref[...]
    b = b_ref[...]
    srow = jnp.zeros((1, HID), jnp.float32)
    qrow = jnp.zeros((1, HID), jnp.float32)
    for g in range(G):
        r = jnp.maximum(pa + be_ref[g], 0.0)
        h2 = jnp.dot(r, w, preferred_element_type=jnp.float32) + b
        h2_ref[g] = h2
        srow = srow + jnp.sum(h2, axis=0, keepdims=True)
        qrow = qrow + jnp.sum(h2 * h2, axis=0, keepdims=True)

    @pl.when(first)
    def _():
        hs_ref[...] = srow
        hq_ref[...] = qrow

    @pl.when(jnp.logical_not(first))
    def _():
        hs_ref[...] = hs_ref[...] + srow
        hq_ref[...] = hq_ref[...] + qrow


def _tc_layer2(P, alpha, beta, W2t, b2):
    nt = NG // TR
    out = pl.pallas_call(
        _k9_body,
        grid=(nt,),
        in_specs=[
            pl.BlockSpec((TR, 2 * HID), lambda i: (i, 0)),
            pl.BlockSpec((1, 2 * HID), lambda i: (0, 0)),
            pl.BlockSpec((G, 1, 2 * HID), lambda i: (0, 0, 0)),
            pl.BlockSpec((2 * HID, HID), lambda i: (0, 0)),
            pl.BlockSpec((1, HID), lambda i: (0, 0)),
        ],
        out_specs=(
            pl.BlockSpec((G, TR, HID), lambda i: (0, i, 0)),
            pl.BlockSpec((1, HID), lambda i: (0, 0)),
            pl.BlockSpec((1, HID), lambda i: (0, 0)),
        ),
        out_shape=(
            jax.ShapeDtypeStruct((G, NG, HID), jnp.float32),
            jax.ShapeDtypeStruct((1, HID), jnp.float32),
            jax.ShapeDtypeStruct((1, HID), jnp.float32),
        ),
    )(P, alpha, beta.reshape(G, 1, 2 * HID), W2t, b2)
    return out[0].reshape(G * NG, HID), out[1], out[2]


def _k10_body(h2_ref, a_ref, c_ref, w_ref, b_ref, h3_ref, hs_ref, hq_ref):
    first = pl.program_id(0) == 0
    t = jnp.maximum(h2_ref[...] * a_ref[...] + c_ref[...], 0.0) * w_ref[...]
    row = jnp.sum(t, axis=1, keepdims=True) + b_ref[...]
    h3_ref[...] = row
    srow = jnp.sum(row, axis=0, keepdims=True)
    qrow = jnp.sum(row * row, axis=0, keepdims=True)

    @pl.when(first)
    def _():
        hs_ref[...] = srow
        hq_ref[...] = qrow

    @pl.when(jnp.logical_not(first))
    def _():
        hs_ref[...] = hs_ref[...] + srow
        hq_ref[...] = hq_ref[...] + qrow


def _tc_layer3(H2, a3, c3, w3, b3):
    nt = (G * NG) // TR
    return pl.pallas_call(
        _k10_body,
        grid=(nt,),
        in_specs=[
            pl.BlockSpec((TR, HID), lambda i: (i, 0)),
            pl.BlockSpec((1, HID), lambda i: (0, 0)),
            pl.BlockSpec((1, HID), lambda i: (0, 0)),
            pl.BlockSpec((1, HID), lambda i: (0, 0)),
            pl.BlockSpec((1, 1), lambda i: (0, 0)),
        ],
        out_specs=(
            pl.BlockSpec((TR, 1), lambda i: (i, 0)),
            pl.BlockSpec((1, 1), lambda i: (0, 0)),
            pl.BlockSpec((1, 1), lambda i: (0, 0)),
        ),
        out_shape=(
            jax.ShapeDtypeStruct((G * NG, 1), jnp.float32),
            jax.ShapeDtypeStruct((1, 1), jnp.float32),
            jax.ShapeDtypeStruct((1, 1), jnp.float32),
        ),
    )(H2, a3, c3, w3, b3)


def _k11_body(h3_ref, e_ref, a_ref, c_ref, out_ref):
    out_ref[...] = h3_ref[...] * a_ref[...] + c_ref[...] + e_ref[...]


def _tc_final(h3, expr, a4, c4):
    nt = (G * NG) // TR
    return pl.pallas_call(
        _k11_body,
        grid=(nt,),
        in_specs=[
            pl.BlockSpec((TR, 1), lambda i: (i, 0)),
            pl.BlockSpec((TR, 1), lambda i: (i, 0)),
            pl.BlockSpec((1, 1), lambda i: (0, 0)),
            pl.BlockSpec((1, 1), lambda i: (0, 0)),
        ],
        out_specs=pl.BlockSpec((TR, 1), lambda i: (i, 0)),
        out_shape=jax.ShapeDtypeStruct((G * NG, 1), jnp.float32),
    )(h3, expr, a4, c4)


def kernel(x, batch, G_sim, G_sim_weight, gene_emb, pert_emb,
           sg_W0, sg_b0, sg_W1, sg_b1,
           rec_W1, rec_b1, rec_W2, rec_b2, rec_W3, rec_b3):
    npad = EPAD - E
    src2 = jnp.concatenate([G_sim[0], jnp.zeros((npad,), G_sim.dtype)]).reshape(-1, ECH)
    dst2 = jnp.concatenate([G_sim[1], jnp.zeros((npad,), G_sim.dtype)]).reshape(-1, ECH)
    ew2 = jnp.concatenate(
        [G_sim_weight, jnp.zeros((npad,), G_sim_weight.dtype)]).reshape(-1, ECH)

    base, p0, bm, bsq = _tc_prep(gene_emb, pert_emb)

    degp = _sc_deg(dst2, ew2)
    dinv = _tc_dinv(degp)
    dinv_col = dinv.reshape(NGP)[:NG].reshape(NG, 1)

    y0 = _tc_rowscale(p0, dinv_col)
    s0 = _sc_edge_scatter(src2, dst2, ew2, y0.reshape(2 * NG, HID))
    pge1 = _tc_sg_linear(s0, p0, dinv_col, sg_W0.T, sg_b0, do_relu=True)
    y1 = _tc_rowscale(pge1, dinv_col)
    s1 = _sc_edge_scatter(src2, dst2, ew2, y1.reshape(2 * NG, HID))
    pge2 = _tc_sg_linear(s1, pge1, dinv_col, sg_W1.T, sg_b1, do_relu=False)

    pertmat = x[:, 1].reshape(G, NG)
    add = _tc_pert_mix(pertmat, pge2)

    am = jnp.mean(add, axis=0, keepdims=True)
    av = jnp.mean(add * add, axis=0, keepdims=True) - am * am
    m1 = bm + am
    v1 = (bsq - bm * bm) + av
    sd1 = jnp.sqrt(v1 + EPS)

    W1s = rec_W1.T / sd1.reshape(HID, 1)
    c1 = (m1 / sd1) @ rec_W1.T
    Q = (add / sd1) @ rec_W1.T + rec_b1
    P, Pm, Psq = _tc_layer1(base, W1s, c1)
    qm = jnp.mean(Q, axis=0, keepdims=True)
    qv = jnp.mean(Q * Q, axis=0, keepdims=True) - qm * qm
    m2 = Pm + qm
    sd2 = jnp.sqrt((Psq - Pm * Pm) + qv + EPS)
    alpha = 1.0 / sd2
    beta = (Q - m2) / sd2

    H2, Hs, Hq = _tc_layer2(P, alpha, beta, rec_W2.T, rec_b2.reshape(1, HID))
    m3 = Hs / (G * NG)
    v3 = Hq / (G * NG) - m3 * m3
    sd3 = jnp.sqrt(v3 + EPS)
    a3 = 1.0 / sd3
    c3 = -m3 / sd3

    h3, hs, hq = _tc_layer3(H2, a3, c3, rec_W3, rec_b3.reshape(1, 1))
    m4 = hs / (G * NG)
    v4 = hq / (G * NG) - m4 * m4
    sd4 = jnp.sqrt(v4 + EPS)

    return _tc_final(h3, x[:, 0:1], 1.0 / sd4, -m4 / sd4)


# speedup vs baseline: 9.1040x; 1.0054x over previous
import functools

import jax
import jax.numpy as jnp
from jax import lax
from jax.experimental import pallas as pl
from jax.experimental.pallas import tpu as pltpu
from jax.experimental.pallas import tpu_sc as plsc

NG = 10000
NGP = 10240
HID = 128
G = 8
E = 320000
CHUNK = 80
EPS = 1e-5
NORM_EPS = 1e-7
LAM = 0.2

_MESH = dict(core_axis_name="c", subcore_axis_name="s")


EPAD = 2560 * 128
ECH = 128
CPTD = 80
CPT = 160
CW = HID // 2
RPT = NGP // 16
CF = 120
CS = 40


def _sc_deg(dst2, ew2):
    @functools.partial(
        pl.kernel,
        out_type=jax.ShapeDtypeStruct((2, NGP), jnp.float32),
        mesh=plsc.VectorSubcoreMesh(**_MESH),
        scratch_types=[
            pltpu.VMEM_SHARED((NGP,), jnp.float32),
            pltpu.VMEM((CPTD, ECH), jnp.int32),
            pltpu.VMEM((CPTD, ECH), jnp.float32),
            pltpu.VMEM((640,), jnp.float32),
            pltpu.SemaphoreType.DMA,
        ],
    )
    def deg_kernel(dst_hbm, ew_hbm, out_hbm, acc_sh, dst_v, ew_v, zero_v, sem):
        c = lax.axis_index("c")
        s = lax.axis_index("s")
        wid = s * 2 + c

        @pl.loop(0, 40)
        def _fill(i):
            zero_v[pl.ds(i * 16, 16)] = jnp.zeros((16,), jnp.float32)

        pltpu.sync_copy(zero_v, acc_sh.at[pl.ds(s * 640, 640)])
        pltpu.sync_copy(dst_hbm.at[pl.ds(wid * CPTD, CPTD)], dst_v)
        pltpu.sync_copy(ew_hbm.at[pl.ds(wid * CPTD, CPTD)], ew_v)
        plsc.subcore_barrier()

        @pl.loop(0, CPTD // 8)
        def _groups(t):
            ds_ = [
                pltpu.async_copy(ew_v.at[t * 8 + u], acc_sh.at[dst_v.at[t * 8 + u]],
                                 sem, add=True)
                for u in range(8)
            ]
            for d in ds_:
                d.wait()

        plsc.subcore_barrier()
        pltpu.sync_copy(acc_sh.at[pl.ds(s * 640, 640)],
                        out_hbm.at[c, pl.ds(s * 640, 640)])

    return deg_kernel(dst2, ew2)


def _sc_edge_scatter(src2, dst2, ew2, table):
    @functools.partial(
        pl.kernel,
        out_type=jax.ShapeDtypeStruct((2, NGP, HID), jnp.float32),
        mesh=plsc.VectorSubcoreMesh(**_MESH),
        scratch_types=[
            pltpu.VMEM_SHARED((NGP, HID), jnp.float32),
            pltpu.VMEM((CF, ECH), jnp.int32),
            pltpu.VMEM((2, ECH), jnp.int32),
            pltpu.VMEM((2, ECH), jnp.float32),
            pltpu.VMEM((2, ECH, HID), jnp.float32),
            [pltpu.SemaphoreType.DMA] * 2,
            [pltpu.SemaphoreType.DMA] * 2,
            [pltpu.SemaphoreType.DMA] * 2,
        ],
    )
    def scat_kernel(src_hbm, dst_hbm, ew_hbm, tab_hbm, out_hbm,
                    acc_sh, src_v, dst_v, ew_v, rows_v, isems, gsems, ssems):
        c = lax.axis_index("c")
        s = lax.axis_index("s")
        base = jnp.where(c == 0, s * CF, 16 * CF + s * CS)
        nc = jnp.where(c == 0, CF, CS)

        @pl.loop(0, ECH)
        def _z(r):
            for k in range(HID // 16):
                rows_v[0, r, pl.ds(k * 16, 16)] = jnp.zeros((16,), jnp.float32)

        for t in range(RPT // ECH):
            pltpu.sync_copy(rows_v.at[0], acc_sh.at[pl.ds(s * RPT + t * ECH, ECH)])
        pltpu.sync_copy(src_hbm.at[pl.ds(base, CS)], src_v.at[pl.ds(0, CS)])

        @pl.when(c == 0)
        def _more():
            pltpu.sync_copy(src_hbm.at[pl.ds(base + CS, CF - CS)],
                            src_v.at[pl.ds(CS, CF - CS)])

        @pl.loop(0, CF)
        def _remap(r):
            @pl.loop(0, ECH // 16)
            def _rb(b):
                sl = pl.ds(b * 16, 16)
                src_v[r, sl] = src_v[r, sl] + c * NG

        plsc.subcore_barrier()

        def fire(j, u):
            pltpu.async_copy(dst_hbm.at[base + j], dst_v.at[u], isems[u])
            pltpu.async_copy(ew_hbm.at[base + j], ew_v.at[u], isems[u])
            pltpu.async_copy(tab_hbm.at[src_v.at[j]], rows_v.at[u], gsems[u])

        def wait_in(u):
            pltpu.make_async_copy(dst_hbm.at[0], dst_v.at[u], isems[u]).wait()
            pltpu.make_async_copy(ew_hbm.at[0], ew_v.at[u], isems[u]).wait()
            pltpu.make_async_copy(tab_hbm.at[src_v.at[0]], rows_v.at[u],
                                  gsems[u]).wait()

        def scale(u):
            @pl.loop(0, ECH // 16)
            def _scale(b):
                wv = ew_v[u, pl.ds(b * 16, 16)]
                for e in range(16):
                    w = wv[e]
                    for k in range(HID // 16):
                        sl = pl.ds(k * 16, 16)
                        rows_v[u, b * 16 + e, sl] = rows_v[u, b * 16 + e, sl] * w

        def scat(u):
            pltpu.async_copy(rows_v.at[u], acc_sh.at[dst_v.at[u]], ssems[u],
                             add=True)

        def wait_scat(u):
            pltpu.make_async_copy(rows_v.at[u], acc_sh.at[dst_v.at[u]],
                                  ssems[u]).wait()

        fire(0, 0)
        fire(1, 1)

        @pl.loop(0, jnp.where(c == 0, CF // 2, CS // 2))
        def _pairs(t):
            j0 = t * 2
            wait_in(0)
            scale(0)
            scat(0)
            wait_in(1)
            scale(1)
            scat(1)
            wait_scat(0)

            @pl.when(j0 + 2 < nc)
            def _():
                fire(j0 + 2, 0)

            wait_scat(1)

            @pl.when(j0 + 3 < nc)
            def _():
                fire(j0 + 3, 1)

        plsc.subcore_barrier()
        pltpu.sync_copy(acc_sh.at[pl.ds(s * RPT, RPT)],
                        out_hbm.at[c, pl.ds(s * RPT, RPT)])

    return scat_kernel(src2, dst2, ew2, table)


def _k1_body(ge_ref, pe_ref, base_ref, p0_ref, bm_ref, bsq_ref):
    ge = ge_ref[...]
    n = jnp.sqrt(jnp.sum(ge * ge, axis=1, keepdims=True))
    r = ge * jnp.minimum(1.0, 1.0 / (n + NORM_EPS))
    m = jnp.mean(r, axis=0, keepdims=True)
    v = jnp.mean(r * r, axis=0, keepdims=True) - m * m
    base = jnp.maximum((r - m) / jnp.sqrt(v + EPS), 0.0)
    base_ref[...] = base
    bm_ref[...] = jnp.mean(base, axis=0, keepdims=True)
    bsq_ref[...] = jnp.mean(base * base, axis=0, keepdims=True)
    pe = pe_ref[...]
    n2 = jnp.sqrt(jnp.sum(pe * pe, axis=1, keepdims=True))
    p0_ref[...] = pe * jnp.minimum(1.0, 1.0 / (n2 + NORM_EPS))


def _tc_prep(gene_emb, pert_emb):
    return pl.pallas_call(
        _k1_body,
        out_shape=(
            jax.ShapeDtypeStruct((NG, HID), jnp.float32),
            jax.ShapeDtypeStruct((NG, HID), jnp.float32),
            jax.ShapeDtypeStruct((1, HID), jnp.float32),
            jax.ShapeDtypeStruct((1, HID), jnp.float32),
        ),
    )(gene_emb, pert_emb)


def _k5_body(sp_ref, xf_ref, dc_ref, w_ref, b_ref, out_ref, *, do_relu):
    dc = dc_ref[...]
    z = (sp_ref[0, :NG] + sp_ref[1, :NG]) * dc + xf_ref[...] * (dc * dc)
    h = jnp.dot(z, w_ref[...], preferred_element_type=jnp.float32) + b_ref[...]
    out_ref[...] = jnp.maximum(h, 0.0) if do_relu else h


def _tc_sg_linear(sp, xf, dinv_col, Wt, b, do_relu):
    return pl.pallas_call(
        functools.partial(_k5_body, do_relu=do_relu),
        out_shape=jax.ShapeDtypeStruct((NG, HID), jnp.float32),
    )(sp, xf, dinv_col, Wt, b.reshape(1, HID))


def _kdinv_body(dp_ref, out_ref):
    out_ref[...] = lax.rsqrt(dp_ref[0:1] + dp_ref[1:2] + 1.0)


def _tc_dinv(degp):
    return pl.pallas_call(
        _kdinv_body,
        out_shape=jax.ShapeDtypeStruct((1, NGP), jnp.float32),
    )(degp)


def _krs_body(t_ref, d_ref, out_ref):
    y = t_ref[...] * d_ref[...]
    out_ref[0] = y
    out_ref[1] = y


def _tc_rowscale(tab, dinv_col):
    return pl.pallas_call(
        _krs_body,
        out_shape=jax.ShapeDtypeStruct((2, NG, HID), jnp.float32),
    )(tab, dinv_col)


def _k7_body(pm_ref, pge_ref, add_ref):
    add_ref[...] = LAM * jnp.dot(pm_ref[...], pge_ref[...],
                                 preferred_element_type=jnp.float32)


def _tc_pert_mix(pertmat, pge2):
    return pl.pallas_call(
        _k7_body,
        out_shape=jax.ShapeDtypeStruct((G, HID), jnp.float32),
    )(pertmat, pge2)


def _k8_body(base_ref, w_ref, c_ref, p_ref, pm_ref, psq_ref):
    p = jnp.dot(base_ref[...], w_ref[...],
                preferred_element_type=jnp.float32) - c_ref[...]
    p_ref[...] = p
    pm_ref[...] = jnp.mean(p, axis=0, keepdims=True)
    psq_ref[...] = jnp.mean(p * p, axis=0, keepdims=True)


def _tc_layer1(base, W1s, c1):
    return pl.pallas_call(
        _k8_body,
        out_shape=(
            jax.ShapeDtypeStruct((NG, 2 * HID), jnp.float32),
            jax.ShapeDtypeStruct((1, 2 * HID), jnp.float32),
            jax.ShapeDtypeStruct((1, 2 * HID), jnp.float32),
        ),
    )(base, W1s, c1)


TR = 1000


def _k9_body(p_ref, al_ref, be_ref, w_ref, b_ref, h2_ref, hs_ref, hq_ref):
    first = pl.program_id(0) == 0
    pa = p_ref[...] * al_ref[...]
    w = w_ref[...]
    b = b_ref[...]
    srow = jnp.zeros((1, HID), jnp.float32)
    qrow = jnp.zeros((1, HID), jnp.float32)
    for g in range(G):
        r = jnp.maximum(pa + be_ref[g], 0.0)
        h2 = jnp.dot(r, w, preferred_element_type=jnp.float32) + b
        h2_ref[g] = h2.astype(jnp.bfloat16)
        srow = srow + jnp.sum(h2, axis=0, keepdims=True)
        qrow = qrow + jnp.sum(h2 * h2, axis=0, keepdims=True)

    @pl.when(first)
    def _():
        hs_ref[...] = srow
        hq_ref[...] = qrow

    @pl.when(jnp.logical_not(first))
    def _():
        hs_ref[...] = hs_ref[...] + srow
        hq_ref[...] = hq_ref[...] + qrow


def _tc_layer2(P, alpha, beta, W2t, b2):
    nt = NG // TR
    out = pl.pallas_call(
        _k9_body,
        grid=(nt,),
        in_specs=[
            pl.BlockSpec((TR, 2 * HID), lambda i: (i, 0)),
            pl.BlockSpec((1, 2 * HID), lambda i: (0, 0)),
            pl.BlockSpec((G, 1, 2 * HID), lambda i: (0, 0, 0)),
            pl.BlockSpec((2 * HID, HID), lambda i: (0, 0)),
            pl.BlockSpec((1, HID), lambda i: (0, 0)),
        ],
        out_specs=(
            pl.BlockSpec((G, TR, HID), lambda i: (0, i, 0)),
            pl.BlockSpec((1, HID), lambda i: (0, 0)),
            pl.BlockSpec((1, HID), lambda i: (0, 0)),
        ),
        out_shape=(
            jax.ShapeDtypeStruct((G, NG, HID), jnp.bfloat16),
            jax.ShapeDtypeStruct((1, HID), jnp.float32),
            jax.ShapeDtypeStruct((1, HID), jnp.float32),
        ),
    )(P, alpha, beta.reshape(G, 1, 2 * HID), W2t, b2)
    return out[0].reshape(G * NG, HID), out[1], out[2]


def _k10_body(h2_ref, a_ref, c_ref, w_ref, b_ref, h3_ref, hs_ref, hq_ref):
    first = pl.program_id(0) == 0
    h2f = h2_ref[...].astype(jnp.float32)
    t = jnp.maximum(h2f * a_ref[...] + c_ref[...], 0.0) * w_ref[...]
    row = jnp.sum(t, axis=1, keepdims=True) + b_ref[...]
    h3_ref[...] = row
    srow = jnp.sum(row, axis=0, keepdims=True)
    qrow = jnp.sum(row * row, axis=0, keepdims=True)

    @pl.when(first)
    def _():
        hs_ref[...] = srow
        hq_ref[...] = qrow

    @pl.when(jnp.logical_not(first))
    def _():
        hs_ref[...] = hs_ref[...] + srow
        hq_ref[...] = hq_ref[...] + qrow


def _tc_layer3(H2, a3, c3, w3, b3):
    nt = (G * NG) // TR
    return pl.pallas_call(
        _k10_body,
        grid=(nt,),
        in_specs=[
            pl.BlockSpec((TR, HID), lambda i: (i, 0)),
            pl.BlockSpec((1, HID), lambda i: (0, 0)),
            pl.BlockSpec((1, HID), lambda i: (0, 0)),
            pl.BlockSpec((1, HID), lambda i: (0, 0)),
            pl.BlockSpec((1, 1), lambda i: (0, 0)),
        ],
        out_specs=(
            pl.BlockSpec((TR, 1), lambda i: (i, 0)),
            pl.BlockSpec((1, 1), lambda i: (0, 0)),
            pl.BlockSpec((1, 1), lambda i: (0, 0)),
        ),
        out_shape=(
            jax.ShapeDtypeStruct((G * NG, 1), jnp.float32),
            jax.ShapeDtypeStruct((1, 1), jnp.float32),
            jax.ShapeDtypeStruct((1, 1), jnp.float32),
        ),
    )(H2, a3, c3, w3, b3)


def _k11_body(h3_ref, e_ref, a_ref, c_ref, out_ref):
    out_ref[...] = h3_ref[...] * a_ref[...] + c_ref[...] + e_ref[...]


def _tc_final(h3, expr, a4, c4):
    nt = (G * NG) // TR
    return pl.pallas_call(
        _k11_body,
        grid=(nt,),
        in_specs=[
            pl.BlockSpec((TR, 1), lambda i: (i, 0)),
            pl.BlockSpec((TR, 1), lambda i: (i, 0)),
            pl.BlockSpec((1, 1), lambda i: (0, 0)),
            pl.BlockSpec((1, 1), lambda i: (0, 0)),
        ],
        out_specs=pl.BlockSpec((TR, 1), lambda i: (i, 0)),
        out_shape=jax.ShapeDtypeStruct((G * NG, 1), jnp.float32),
    )(h3, expr, a4, c4)


def kernel(x, batch, G_sim, G_sim_weight, gene_emb, pert_emb,
           sg_W0, sg_b0, sg_W1, sg_b1,
           rec_W1, rec_b1, rec_W2, rec_b2, rec_W3, rec_b3):
    npad = EPAD - E
    src2 = jnp.concatenate([G_sim[0], jnp.zeros((npad,), G_sim.dtype)]).reshape(-1, ECH)
    dst2 = jnp.concatenate([G_sim[1], jnp.zeros((npad,), G_sim.dtype)]).reshape(-1, ECH)
    ew2 = jnp.concatenate(
        [G_sim_weight, jnp.zeros((npad,), G_sim_weight.dtype)]).reshape(-1, ECH)

    base, p0, bm, bsq = _tc_prep(gene_emb, pert_emb)

    degp = _sc_deg(dst2, ew2)
    dinv = _tc_dinv(degp)
    dinv_col = dinv.reshape(NGP)[:NG].reshape(NG, 1)

    y0 = _tc_rowscale(p0, dinv_col)
    s0 = _sc_edge_scatter(src2, dst2, ew2, y0.reshape(2 * NG, HID))
    pge1 = _tc_sg_linear(s0, p0, dinv_col, sg_W0.T, sg_b0, do_relu=True)
    y1 = _tc_rowscale(pge1, dinv_col)
    s1 = _sc_edge_scatter(src2, dst2, ew2, y1.reshape(2 * NG, HID))
    pge2 = _tc_sg_linear(s1, pge1, dinv_col, sg_W1.T, sg_b1, do_relu=False)

    pertmat = x[:, 1].reshape(G, NG)
    add = _tc_pert_mix(pertmat, pge2)

    am = jnp.mean(add, axis=0, keepdims=True)
    av = jnp.mean(add * add, axis=0, keepdims=True) - am * am
    m1 = bm + am
    v1 = (bsq - bm * bm) + av
    sd1 = jnp.sqrt(v1 + EPS)

    W1s = rec_W1.T / sd1.reshape(HID, 1)
    c1 = (m1 / sd1) @ rec_W1.T
    Q = (add / sd1) @ rec_W1.T + rec_b1
    P, Pm, Psq = _tc_layer1(base, W1s, c1)
    qm = jnp.mean(Q, axis=0, keepdims=True)
    qv = jnp.mean(Q * Q, axis=0, keepdims=True) - qm * qm
    m2 = Pm + qm
    sd2 = jnp.sqrt((Psq - Pm * Pm) + qv + EPS)
    alpha = 1.0 / sd2
    beta = (Q - m2) / sd2

    H2, Hs, Hq = _tc_layer2(P, alpha, beta, rec_W2.T, rec_b2.reshape(1, HID))
    m3 = Hs / (G * NG)
    v3 = Hq / (G * NG) - m3 * m3
    sd3 = jnp.sqrt(v3 + EPS)
    a3 = 1.0 / sd3
    c3 = -m3 / sd3

    h3, hs, hq = _tc_layer3(H2, a3, c3, rec_W3, rec_b3.reshape(1, 1))
    m4 = hs / (G * NG)
    v4 = hq / (G * NG) - m4 * m4
    sd4 = jnp.sqrt(v4 + EPS)

    return _tc_final(h3, x[:, 0:1], 1.0 / sd4, -m4 / sd4)
